# Initial kernel scaffold; baseline (speedup 1.0000x reference)
import jax
import jax.numpy as jnp
from jax.experimental import pallas as pl


def kernel(x, edge_index, W1, b1, W2, b2):
    raise NotImplementedError("write your pallas kernel here")


# speedup vs baseline: 29.3343x; 29.3343x over previous
import functools

import jax
import jax.numpy as jnp
from jax import lax
from jax.experimental import pallas as pl
from jax.experimental.pallas import tpu as pltpu
from jax.experimental.pallas import tpu_sc as plsc

N = 10000
E = 320000
F_IN = 128
H = 16
C = 100

NC = 2
NS = 16
NW = NC * NS
EPT = E // NW
BB = 80
NB = EPT // BB
NPAD = 10240
RPS = NPAD // NS
BLK = 1000

_mesh = plsc.VectorSubcoreMesh(core_axis_name="c", subcore_axis_name="s")
_sc_params = pltpu.CompilerParams(use_tc_tiling_on_sc=False)


@functools.partial(
    pl.kernel,
    out_type=jax.ShapeDtypeStruct((NC, NPAD), jnp.float32),
    mesh=_mesh,
    scratch_types=[
        pltpu.VMEM((NB, BB), jnp.int32),
        pltpu.VMEM((BB,), jnp.float32),
        pltpu.VMEM((RPS,), jnp.float32),
        pltpu.VMEM_SHARED((NPAD,), jnp.float32),
    ],
    compiler_params=_sc_params,
)
def _sc_degree(dst_hbm, out_hbm, idx_v, ones_v, buf_v, tbl):
    c = lax.axis_index("c")
    s = lax.axis_index("s")
    wid = c * NS + s

    def _fill_ones(i, _):
        ones_v[pl.ds(i * 16, 16)] = jnp.ones((16,), jnp.float32)
        return 0

    lax.fori_loop(0, BB // 16, _fill_ones, 0)

    def _fill_zeros(i, _):
        buf_v[pl.ds(i * 16, 16)] = jnp.zeros((16,), jnp.float32)
        return 0

    lax.fori_loop(0, RPS // 16, _fill_zeros, 0)

    pltpu.sync_copy(buf_v, tbl.at[pl.ds(s * RPS, RPS)])
    pltpu.sync_copy(dst_hbm.at[wid], idx_v)
    plsc.subcore_barrier()

    def _step(j, _):
        pltpu.sync_copy(ones_v, tbl.at[idx_v.at[j]], add=True)
        return 0

    lax.fori_loop(0, NB, _step, 0)

    plsc.subcore_barrier()
    pltpu.sync_copy(tbl.at[pl.ds(s * RPS, RPS)], buf_v)
    pltpu.sync_copy(buf_v, out_hbm.at[c, pl.ds(s * RPS, RPS)])


@functools.partial(
    pl.kernel,
    out_type=jax.ShapeDtypeStruct((NC, NPAD, H), jnp.float32),
    mesh=_mesh,
    scratch_types=[
        pltpu.VMEM((NB, BB), jnp.int32),
        pltpu.VMEM((NB, BB), jnp.int32),
        pltpu.VMEM((BB, H), jnp.float32),
        pltpu.VMEM((RPS, H), jnp.float32),
        pltpu.VMEM_SHARED((NPAD, H), jnp.float32),
        pltpu.SemaphoreType.DMA,
    ],
    compiler_params=_sc_params,
)
def _sc_aggregate(xs_hbm, src_hbm, dst_hbm, out_hbm,
                  src_v, dst_v, rows_v, buf_v, tbl, sem):
    c = lax.axis_index("c")
    s = lax.axis_index("s")
    wid = c * NS + s

    def _fill_zeros(i, _):
        buf_v[i] = jnp.zeros((H,), jnp.float32)
        return 0

    lax.fori_loop(0, RPS, _fill_zeros, 0)

    pltpu.sync_copy(buf_v, tbl.at[pl.ds(s * RPS, RPS)])
    pltpu.sync_copy(src_hbm.at[wid], src_v)
    pltpu.sync_copy(dst_hbm.at[wid], dst_v)
    plsc.subcore_barrier()

    def _step(j, _):
        pltpu.async_copy(xs_hbm.at[src_v.at[j]], rows_v, sem).wait()
        pltpu.sync_copy(rows_v, tbl.at[dst_v.at[j]], add=True)
        return 0

    lax.fori_loop(0, NB, _step, 0)

    plsc.subcore_barrier()
    pltpu.sync_copy(tbl.at[pl.ds(s * RPS, RPS)], buf_v)
    pltpu.sync_copy(buf_v, out_hbm.at[c, pl.ds(s * RPS, RPS)])


def _tc_layer1_body(x_ref, w1_ref, deg_ref, norm_ref, xs1_ref):
    hw = jnp.dot(x_ref[...], w1_ref[...], preferred_element_type=jnp.float32)
    nrm = lax.rsqrt(1.0 + deg_ref[0] + deg_ref[1])
    norm_ref[...] = nrm
    xs1_ref[...] = hw * nrm


_tc_layer1 = pl.pallas_call(
    _tc_layer1_body,
    grid=(N // BLK,),
    in_specs=[
        pl.BlockSpec((BLK, F_IN), lambda j: (j, 0)),
        pl.BlockSpec((F_IN, H), lambda j: (0, 0)),
        pl.BlockSpec((NC, BLK, 1), lambda j: (0, j, 0)),
    ],
    out_specs=[
        pl.BlockSpec((BLK, 1), lambda j: (j, 0)),
        pl.BlockSpec((BLK, H), lambda j: (j, 0)),
    ],
    out_shape=[
        jax.ShapeDtypeStruct((N, 1), jnp.float32),
        jax.ShapeDtypeStruct((N, H), jnp.float32),
    ],
)


def _tc_mid_body(agg_ref, xs1_ref, norm_ref, b1_ref, xs2_ref):
    tot = agg_ref[0] + agg_ref[1] + xs1_ref[...]
    pre = tot * norm_ref[...] + b1_ref[...]
    xs2_ref[...] = jnp.maximum(pre, 0.0) * norm_ref[...]


_tc_mid = pl.pallas_call(
    _tc_mid_body,
    grid=(N // BLK,),
    in_specs=[
        pl.BlockSpec((NC, BLK, H), lambda j: (0, j, 0)),
        pl.BlockSpec((BLK, H), lambda j: (j, 0)),
        pl.BlockSpec((BLK, 1), lambda j: (j, 0)),
        pl.BlockSpec((1, H), lambda j: (0, 0)),
    ],
    out_specs=pl.BlockSpec((BLK, H), lambda j: (j, 0)),
    out_shape=jax.ShapeDtypeStruct((N, H), jnp.float32),
)


def _tc_out_body(agg_ref, xs2_ref, norm_ref, w2_ref, b2_ref, out_ref):
    z = (agg_ref[0] + agg_ref[1] + xs2_ref[...]) * norm_ref[...]
    logits = jnp.dot(z, w2_ref[...], preferred_element_type=jnp.float32)
    logits = logits + b2_ref[...]
    m = jnp.max(logits, axis=1, keepdims=True)
    shifted = logits - m
    lse = jnp.log(jnp.sum(jnp.exp(shifted), axis=1, keepdims=True))
    out_ref[...] = shifted - lse


_tc_out = pl.pallas_call(
    _tc_out_body,
    grid=(N // BLK,),
    in_specs=[
        pl.BlockSpec((NC, BLK, H), lambda j: (0, j, 0)),
        pl.BlockSpec((BLK, H), lambda j: (j, 0)),
        pl.BlockSpec((BLK, 1), lambda j: (j, 0)),
        pl.BlockSpec((H, C), lambda j: (0, 0)),
        pl.BlockSpec((1, C), lambda j: (0, 0)),
    ],
    out_specs=pl.BlockSpec((BLK, C), lambda j: (j, 0)),
    out_shape=jax.ShapeDtypeStruct((N, C), jnp.float32),
)


def kernel(x, edge_index, W1, b1, W2, b2):
    src3 = edge_index[0].reshape(NW, NB, BB)
    dst3 = edge_index[1].reshape(NW, NB, BB)

    degp = _sc_degree(dst3)
    norm, xs1 = _tc_layer1(x, W1, degp[:, :, None])
    a1 = _sc_aggregate(xs1, src3, dst3)
    xs2 = _tc_mid(a1, xs1, norm, b1.reshape(1, H))
    a2 = _sc_aggregate(xs2, src3, dst3)
    return _tc_out(a2, xs2, norm, W2, b2.reshape(1, C))


# speedup vs baseline: 53.6671x; 1.8295x over previous
import functools

import jax
import jax.numpy as jnp
from jax import lax
from jax.experimental import pallas as pl
from jax.experimental.pallas import tpu as pltpu
from jax.experimental.pallas import tpu_sc as plsc

N = 10000
E = 320000
F_IN = 128
H = 16
C = 100

NC = 2
NS = 16
NW = NC * NS
EPT = E // NW
BB = 80
NB = EPT // BB
NPAD = 10240
RPS = NPAD // NS
BLK = 1000

_mesh = plsc.VectorSubcoreMesh(core_axis_name="c", subcore_axis_name="s")
_sc_params = pltpu.CompilerParams(use_tc_tiling_on_sc=False)


@functools.partial(
    pl.kernel,
    out_type=jax.ShapeDtypeStruct((NC, NPAD), jnp.float32),
    mesh=_mesh,
    scratch_types=[
        pltpu.VMEM((NB, BB), jnp.int32),
        pltpu.VMEM((BB,), jnp.float32),
        pltpu.VMEM((RPS,), jnp.float32),
        pltpu.VMEM_SHARED((NPAD,), jnp.float32),
    ],
    compiler_params=_sc_params,
)
def _sc_degree(dst_hbm, out_hbm, idx_v, ones_v, buf_v, tbl):
    c = lax.axis_index("c")
    s = lax.axis_index("s")
    wid = c * NS + s

    def _fill_ones(i, _):
        ones_v[pl.ds(i * 16, 16)] = jnp.ones((16,), jnp.float32)
        return 0

    lax.fori_loop(0, BB // 16, _fill_ones, 0)

    def _fill_zeros(i, _):
        buf_v[pl.ds(i * 16, 16)] = jnp.zeros((16,), jnp.float32)
        return 0

    lax.fori_loop(0, RPS // 16, _fill_zeros, 0)

    pltpu.sync_copy(buf_v, tbl.at[pl.ds(s * RPS, RPS)])
    pltpu.sync_copy(dst_hbm.at[wid], idx_v)
    plsc.subcore_barrier()

    def _step(j, _):
        pltpu.sync_copy(ones_v, tbl.at[idx_v.at[j]], add=True)
        return 0

    lax.fori_loop(0, NB, _step, 0)

    plsc.subcore_barrier()
    pltpu.sync_copy(tbl.at[pl.ds(s * RPS, RPS)], buf_v)
    pltpu.sync_copy(buf_v, out_hbm.at[c, pl.ds(s * RPS, RPS)])


@functools.partial(
    pl.kernel,
    out_type=jax.ShapeDtypeStruct((NC, NPAD, H), jnp.float32),
    mesh=_mesh,
    scratch_types=[
        pltpu.VMEM((NB, BB), jnp.int32),
        pltpu.VMEM((NB, BB), jnp.int32),
        pltpu.VMEM((2, BB, H), jnp.float32),
        pltpu.VMEM((RPS, H), jnp.float32),
        pltpu.VMEM_SHARED((NPAD, H), jnp.float32),
        pltpu.VMEM_SHARED((NPAD, H), jnp.float32),
        pltpu.SemaphoreType.DMA,
        pltpu.SemaphoreType.DMA,
    ],
    compiler_params=_sc_params,
)
def _sc_aggregate(xs_hbm, src_hbm, dst_hbm, out_hbm,
                  src_v, dst_v, rows_v, buf_v, xs_sh, tbl, sem0, sem1):
    c = lax.axis_index("c")
    s = lax.axis_index("s")
    wid = c * NS + s

    pltpu.sync_copy(xs_hbm.at[pl.ds(s * RPS, RPS)], buf_v)
    pltpu.sync_copy(buf_v, xs_sh.at[pl.ds(s * RPS, RPS)])

    def _fill_zeros(i, _):
        buf_v[i] = jnp.zeros((H,), jnp.float32)
        return 0

    lax.fori_loop(0, RPS, _fill_zeros, 0)

    pltpu.sync_copy(buf_v, tbl.at[pl.ds(s * RPS, RPS)])
    pltpu.sync_copy(src_hbm.at[wid], src_v)
    pltpu.sync_copy(dst_hbm.at[wid], dst_v)
    plsc.subcore_barrier()

    def _gather(j, b, sem):
        return pltpu.async_copy(xs_sh.at[src_v.at[j]], rows_v.at[b], sem)

    def _gwait(b, sem):
        pltpu.make_async_copy(xs_sh.at[src_v.at[0]], rows_v.at[b], sem).wait()

    def _scatter(j, b):
        pltpu.sync_copy(rows_v.at[b], tbl.at[dst_v.at[j]], add=True)

    _gather(0, 0, sem0)

    def _step(i, _):
        j0 = 2 * i
        _gather(j0 + 1, 1, sem1)
        _gwait(0, sem0)
        _scatter(j0, 0)
        _gather(j0 + 2, 0, sem0)
        _gwait(1, sem1)
        _scatter(j0 + 1, 1)
        return 0

    lax.fori_loop(0, (NB - 1) // 2, _step, 0)
    _gwait(0, sem0)
    _scatter(NB - 1, 0)

    plsc.subcore_barrier()
    pltpu.sync_copy(tbl.at[pl.ds(s * RPS, RPS)], buf_v)
    pltpu.sync_copy(buf_v, out_hbm.at[c, pl.ds(s * RPS, RPS)])


def _tc_layer1_body(x_ref, w1_ref, deg_ref, norm_ref, xs1_ref):
    hw = jnp.dot(x_ref[...], w1_ref[...], preferred_element_type=jnp.float32)
    nrm = lax.rsqrt(1.0 + deg_ref[0] + deg_ref[1])
    norm_ref[...] = nrm
    xs1_ref[...] = hw * nrm


_tc_layer1 = pl.pallas_call(
    _tc_layer1_body,
    grid=(N // BLK,),
    in_specs=[
        pl.BlockSpec((BLK, F_IN), lambda j: (j, 0)),
        pl.BlockSpec((F_IN, H), lambda j: (0, 0)),
        pl.BlockSpec((NC, BLK, 1), lambda j: (0, j, 0)),
    ],
    out_specs=[
        pl.BlockSpec((BLK, 1), lambda j: (j, 0)),
        pl.BlockSpec((BLK, H), lambda j: (j, 0)),
    ],
    out_shape=[
        jax.ShapeDtypeStruct((N, 1), jnp.float32),
        jax.ShapeDtypeStruct((NPAD, H), jnp.float32),
    ],
)


def _tc_mid_body(agg_ref, xs1_ref, norm_ref, b1_ref, xs2_ref):
    tot = agg_ref[0] + agg_ref[1] + xs1_ref[...]
    pre = tot * norm_ref[...] + b1_ref[...]
    xs2_ref[...] = jnp.maximum(pre, 0.0) * norm_ref[...]


_tc_mid = pl.pallas_call(
    _tc_mid_body,
    grid=(N // BLK,),
    in_specs=[
        pl.BlockSpec((NC, BLK, H), lambda j: (0, j, 0)),
        pl.BlockSpec((BLK, H), lambda j: (j, 0)),
        pl.BlockSpec((BLK, 1), lambda j: (j, 0)),
        pl.BlockSpec((1, H), lambda j: (0, 0)),
    ],
    out_specs=pl.BlockSpec((BLK, H), lambda j: (j, 0)),
    out_shape=jax.ShapeDtypeStruct((NPAD, H), jnp.float32),
)


def _tc_out_body(agg_ref, xs2_ref, norm_ref, w2_ref, b2_ref, out_ref):
    z = (agg_ref[0] + agg_ref[1] + xs2_ref[...]) * norm_ref[...]
    logits = jnp.dot(z, w2_ref[...], preferred_element_type=jnp.float32)
    logits = logits + b2_ref[...]
    m = jnp.max(logits, axis=1, keepdims=True)
    shifted = logits - m
    lse = jnp.log(jnp.sum(jnp.exp(shifted), axis=1, keepdims=True))
    out_ref[...] = shifted - lse


_tc_out = pl.pallas_call(
    _tc_out_body,
    grid=(N // BLK,),
    in_specs=[
        pl.BlockSpec((NC, BLK, H), lambda j: (0, j, 0)),
        pl.BlockSpec((BLK, H), lambda j: (j, 0)),
        pl.BlockSpec((BLK, 1), lambda j: (j, 0)),
        pl.BlockSpec((H, C), lambda j: (0, 0)),
        pl.BlockSpec((1, C), lambda j: (0, 0)),
    ],
    out_specs=pl.BlockSpec((BLK, C), lambda j: (j, 0)),
    out_shape=jax.ShapeDtypeStruct((N, C), jnp.float32),
)


def kernel(x, edge_index, W1, b1, W2, b2):
    src3 = edge_index[0].reshape(NW, NB, BB)
    dst3 = edge_index[1].reshape(NW, NB, BB)

    degp = _sc_degree(dst3)
    norm, xs1 = _tc_layer1(x, W1, degp[:, :, None])
    a1 = _sc_aggregate(xs1, src3, dst3)
    xs2 = _tc_mid(a1, xs1, norm, b1.reshape(1, H))
    a2 = _sc_aggregate(xs2, src3, dst3)
    return _tc_out(a2, xs2, norm, W2, b2.reshape(1, C))


# speedup vs baseline: 55.8434x; 1.0406x over previous
import functools

import jax
import jax.numpy as jnp
from jax import lax
from jax.experimental import pallas as pl
from jax.experimental.pallas import tpu as pltpu
from jax.experimental.pallas import tpu_sc as plsc

N = 10000
E = 320000
F_IN = 128
H = 16
C = 100

NC = 2
NS = 16
NW = NC * NS
EPT = E // NW
BB = 80
NB = EPT // BB
NPAD = 10240
RPS = NPAD // NS
BLK = 1000
DEPTH = 4

_mesh = plsc.VectorSubcoreMesh(core_axis_name="c", subcore_axis_name="s")
_sc_params = pltpu.CompilerParams(use_tc_tiling_on_sc=False)


@functools.partial(
    pl.kernel,
    out_type=jax.ShapeDtypeStruct((NC, NPAD), jnp.float32),
    mesh=_mesh,
    scratch_types=[
        pltpu.VMEM((NB, BB), jnp.int32),
        pltpu.VMEM((BB,), jnp.float32),
        pltpu.VMEM((RPS,), jnp.float32),
        pltpu.VMEM_SHARED((NPAD,), jnp.float32),
        pltpu.SemaphoreType.DMA,
    ],
    compiler_params=_sc_params,
)
def _sc_degree(dst_hbm, zeros_hbm, out_hbm, idx_v, ones_v, buf_v, tbl, sem):
    c = lax.axis_index("c")
    s = lax.axis_index("s")
    wid = c * NS + s

    def _fill_ones(i, _):
        ones_v[pl.ds(i * 16, 16)] = jnp.ones((16,), jnp.float32)
        return 0

    lax.fori_loop(0, BB // 16, _fill_ones, 0)

    pltpu.sync_copy(zeros_hbm, buf_v)
    pltpu.sync_copy(buf_v, tbl.at[pl.ds(s * RPS, RPS)])
    pltpu.sync_copy(dst_hbm.at[wid], idx_v)
    plsc.subcore_barrier()

    def _fire(j, _):
        pltpu.async_copy(ones_v, tbl.at[idx_v.at[j]], sem, add=True)
        return 0

    lax.fori_loop(0, NB, _fire, 0)

    def _drain(j, _):
        pltpu.make_async_copy(ones_v, tbl.at[idx_v.at[0]], sem).wait()
        return 0

    lax.fori_loop(0, NB, _drain, 0)

    plsc.subcore_barrier()
    pltpu.sync_copy(tbl.at[pl.ds(s * RPS, RPS)], buf_v)
    pltpu.sync_copy(buf_v, out_hbm.at[c, pl.ds(s * RPS, RPS)])


@functools.partial(
    pl.kernel,
    out_type=jax.ShapeDtypeStruct((NC, NPAD, H), jnp.float32),
    mesh=_mesh,
    scratch_types=(
        [
            pltpu.VMEM((NB, BB), jnp.int32),
            pltpu.VMEM((NB, BB), jnp.int32),
            pltpu.VMEM((DEPTH, BB, H), jnp.float32),
            pltpu.VMEM((RPS, H), jnp.float32),
            pltpu.VMEM((RPS, H), jnp.float32),
            pltpu.VMEM_SHARED((NPAD, H), jnp.float32),
            pltpu.VMEM_SHARED((NPAD, H), jnp.float32),
        ]
        + [pltpu.SemaphoreType.DMA] * (2 * DEPTH + 1)
    ),
    compiler_params=_sc_params,
)
def _sc_aggregate(xs_hbm, src_hbm, dst_hbm, zeros_hbm, out_hbm,
                  src_v, dst_v, rows_v, sbuf_v, zbuf_v, xs_sh, tbl,
                  g0, g1, g2, g3, s0, s1, s2, s3, stg):
    c = lax.axis_index("c")
    s = lax.axis_index("s")
    wid = c * NS + s
    gsem = (g0, g1, g2, g3)
    ssem = (s0, s1, s2, s3)

    stage = pltpu.async_copy(xs_hbm.at[pl.ds(s * RPS, RPS)], sbuf_v, stg)
    pltpu.sync_copy(src_hbm.at[wid], src_v)
    pltpu.sync_copy(dst_hbm.at[wid], dst_v)
    pltpu.sync_copy(zeros_hbm, zbuf_v)
    pltpu.sync_copy(zbuf_v, tbl.at[pl.ds(s * RPS, RPS)])
    stage.wait()
    pltpu.sync_copy(sbuf_v, xs_sh.at[pl.ds(s * RPS, RPS)])
    plsc.subcore_barrier()

    def _gather(j, b):
        pltpu.async_copy(xs_sh.at[src_v.at[j]], rows_v.at[b], gsem[b])

    def _gwait(b):
        pltpu.make_async_copy(
            xs_sh.at[src_v.at[0]], rows_v.at[b], gsem[b]).wait()

    def _scatter(j, b):
        pltpu.async_copy(rows_v.at[b], tbl.at[dst_v.at[j]], ssem[b], add=True)

    def _swait(b):
        pltpu.make_async_copy(
            rows_v.at[b], tbl.at[dst_v.at[0]], ssem[b]).wait()

    for b in range(DEPTH):
        _gather(b, b)

    def _round(i, _):
        for b in range(DEPTH):
            j = DEPTH * i + b
            _gwait(b)
            _scatter(j, b)
            _swait(b)
            _gather(j + DEPTH, b)
        return 0

    lax.fori_loop(0, 30, _round, 0)
    for b in range(DEPTH):
        j = 120 + b
        _gwait(b)
        _scatter(j, b)
        _swait(b)
        if b == 0:
            _gather(124, 0)
    _gwait(0)
    _scatter(124, 0)
    _swait(0)

    plsc.subcore_barrier()
    pltpu.sync_copy(tbl.at[pl.ds(s * RPS, RPS)], zbuf_v)
    pltpu.sync_copy(zbuf_v, out_hbm.at[c, pl.ds(s * RPS, RPS)])


def _tc_matmul_body(x_ref, w1_ref, hw_ref):
    hw_ref[...] = jnp.dot(x_ref[...], w1_ref[...],
                          preferred_element_type=jnp.float32)


_tc_matmul = pl.pallas_call(
    _tc_matmul_body,
    grid=(N // BLK,),
    in_specs=[
        pl.BlockSpec((BLK, F_IN), lambda j: (j, 0)),
        pl.BlockSpec((F_IN, H), lambda j: (0, 0)),
    ],
    out_specs=pl.BlockSpec((BLK, H), lambda j: (j, 0)),
    out_shape=jax.ShapeDtypeStruct((N, H), jnp.float32),
)


def _tc_scale_body(deg_ref, hw_ref, norm_ref, xs1_ref):
    nrm = lax.rsqrt(1.0 + deg_ref[0] + deg_ref[1])
    norm_ref[...] = nrm
    xs1_ref[...] = hw_ref[...] * nrm


_tc_scale = pl.pallas_call(
    _tc_scale_body,
    grid=(N // BLK,),
    in_specs=[
        pl.BlockSpec((NC, BLK, 1), lambda j: (0, j, 0)),
        pl.BlockSpec((BLK, H), lambda j: (j, 0)),
    ],
    out_specs=[
        pl.BlockSpec((BLK, 1), lambda j: (j, 0)),
        pl.BlockSpec((BLK, H), lambda j: (j, 0)),
    ],
    out_shape=[
        jax.ShapeDtypeStruct((N, 1), jnp.float32),
        jax.ShapeDtypeStruct((NPAD, H), jnp.float32),
    ],
)


def _tc_mid_body(agg_ref, xs1_ref, norm_ref, b1_ref, xs2_ref):
    tot = agg_ref[0] + agg_ref[1] + xs1_ref[...]
    pre = tot * norm_ref[...] + b1_ref[...]
    xs2_ref[...] = jnp.maximum(pre, 0.0) * norm_ref[...]


_tc_mid = pl.pallas_call(
    _tc_mid_body,
    grid=(N // BLK,),
    in_specs=[
        pl.BlockSpec((NC, BLK, H), lambda j: (0, j, 0)),
        pl.BlockSpec((BLK, H), lambda j: (j, 0)),
        pl.BlockSpec((BLK, 1), lambda j: (j, 0)),
        pl.BlockSpec((1, H), lambda j: (0, 0)),
    ],
    out_specs=pl.BlockSpec((BLK, H), lambda j: (j, 0)),
    out_shape=jax.ShapeDtypeStruct((NPAD, H), jnp.float32),
)


def _tc_out_body(agg_ref, xs2_ref, norm_ref, w2_ref, b2_ref, out_ref):
    z = (agg_ref[0] + agg_ref[1] + xs2_ref[...]) * norm_ref[...]
    logits = jnp.dot(z, w2_ref[...], preferred_element_type=jnp.float32)
    logits = logits + b2_ref[...]
    m = jnp.max(logits, axis=1, keepdims=True)
    shifted = logits - m
    lse = jnp.log(jnp.sum(jnp.exp(shifted), axis=1, keepdims=True))
    out_ref[...] = shifted - lse


_tc_out = pl.pallas_call(
    _tc_out_body,
    grid=(N // BLK,),
    in_specs=[
        pl.BlockSpec((NC, BLK, H), lambda j: (0, j, 0)),
        pl.BlockSpec((BLK, H), lambda j: (j, 0)),
        pl.BlockSpec((BLK, 1), lambda j: (j, 0)),
        pl.BlockSpec((H, C), lambda j: (0, 0)),
        pl.BlockSpec((1, C), lambda j: (0, 0)),
    ],
    out_specs=pl.BlockSpec((BLK, C), lambda j: (j, 0)),
    out_shape=jax.ShapeDtypeStruct((N, C), jnp.float32),
)


def kernel(x, edge_index, W1, b1, W2, b2):
    src3 = edge_index[0].reshape(NW, NB, BB)
    dst3 = edge_index[1].reshape(NW, NB, BB)
    z1 = jnp.zeros((RPS,), jnp.float32)
    z2 = jnp.zeros((RPS, H), jnp.float32)

    degp = _sc_degree(dst3, z1)
    hw = _tc_matmul(x, W1)
    norm, xs1 = _tc_scale(degp[:, :, None], hw)
    a1 = _sc_aggregate(xs1, src3, dst3, z2)
    xs2 = _tc_mid(a1, xs1, norm, b1.reshape(1, H))
    a2 = _sc_aggregate(xs2, src3, dst3, z2)
    return _tc_out(a2, xs2, norm, W2, b2.reshape(1, C))


# speedup vs baseline: 57.1822x; 1.0240x over previous
import functools

import jax
import jax.numpy as jnp
from jax import lax
from jax.experimental import pallas as pl
from jax.experimental.pallas import tpu as pltpu
from jax.experimental.pallas import tpu_sc as plsc

N = 10000
E = 320000
F_IN = 128
H = 16
C = 100

NC = 2
NS = 16
NW = NC * NS
EPT = E // NW
BB = 80
NB = EPT // BB
NPAD = 10240
RPS = NPAD // NS
BLK = 2000
DEPTH = 4

_mesh = plsc.VectorSubcoreMesh(core_axis_name="c", subcore_axis_name="s")
_sc_params = pltpu.CompilerParams(use_tc_tiling_on_sc=False)


@functools.partial(
    pl.kernel,
    out_type=jax.ShapeDtypeStruct((NC, NPAD, 1), jnp.float32),
    mesh=_mesh,
    scratch_types=[
        pltpu.VMEM((NB, BB), jnp.int32),
        pltpu.VMEM((BB, 1), jnp.float32),
        pltpu.VMEM((RPS, 1), jnp.float32),
        pltpu.VMEM_SHARED((NPAD, 1), jnp.float32),
        pltpu.SemaphoreType.DMA,
        pltpu.SemaphoreType.DMA,
    ],
    compiler_params=_sc_params,
)
def _sc_degree(dst_hbm, ones_hbm, zeros_hbm, out_hbm,
               idx_v, ones_v, buf_v, tbl, sem, isem):
    c = lax.axis_index("c")
    s = lax.axis_index("s")
    wid = c * NS + s
    base = wid * EPT

    def _ifire(j, _):
        pltpu.async_copy(dst_hbm.at[pl.ds(base + j * BB, BB)],
                         idx_v.at[j], isem)
        return 0

    lax.fori_loop(0, NB, _ifire, 0)

    pltpu.sync_copy(ones_hbm, ones_v)
    pltpu.sync_copy(zeros_hbm, buf_v)
    pltpu.sync_copy(buf_v, tbl.at[pl.ds(s * RPS, RPS)])

    def _idrain(j, _):
        pltpu.make_async_copy(dst_hbm.at[pl.ds(0, BB)],
                              idx_v.at[0], isem).wait()
        return 0

    lax.fori_loop(0, NB, _idrain, 0)
    plsc.subcore_barrier()

    def _fire(j, _):
        pltpu.async_copy(ones_v, tbl.at[idx_v.at[j]], sem, add=True)
        return 0

    lax.fori_loop(0, NB, _fire, 0)

    def _drain(j, _):
        pltpu.make_async_copy(ones_v, tbl.at[idx_v.at[0]], sem).wait()
        return 0

    lax.fori_loop(0, NB, _drain, 0)

    plsc.subcore_barrier()
    pltpu.sync_copy(tbl.at[pl.ds(s * RPS, RPS)], buf_v)
    pltpu.sync_copy(buf_v, out_hbm.at[c, pl.ds(s * RPS, RPS)])


@functools.partial(
    pl.kernel,
    out_type=jax.ShapeDtypeStruct((NC, NPAD, H), jnp.float32),
    mesh=_mesh,
    scratch_types=(
        [
            pltpu.VMEM((EPT,), jnp.int32),
            pltpu.VMEM((NB, BB), jnp.int32),
            pltpu.VMEM((DEPTH, BB, H), jnp.float32),
            pltpu.VMEM((RPS, H), jnp.float32),
            pltpu.VMEM_SHARED((NPAD, H), jnp.float32),
        ]
        + [pltpu.SemaphoreType.DMA] * (2 * DEPTH + 1)
    ),
    compiler_params=_sc_params,
)
def _sc_aggregate(xs_hbm, src_hbm, dst_hbm, zeros_hbm, out_hbm,
                  src_v, dst_v, rows_v, zbuf_v, tbl,
                  g0, g1, g2, g3, s0, s1, s2, s3, isem):
    c = lax.axis_index("c")
    s = lax.axis_index("s")
    wid = c * NS + s
    base = wid * EPT
    gsem = (g0, g1, g2, g3)
    ssem = (s0, s1, s2, s3)

    pltpu.async_copy(src_hbm.at[pl.ds(base, EPT)], src_v, isem)

    def _ifire(j, _):
        pltpu.async_copy(dst_hbm.at[pl.ds(base + j * BB, BB)],
                         dst_v.at[j], isem)
        return 0

    lax.fori_loop(0, NB, _ifire, 0)

    pltpu.sync_copy(zeros_hbm, zbuf_v)
    pltpu.sync_copy(zbuf_v, tbl.at[pl.ds(s * RPS, RPS)])

    def _idrain(j, _):
        pltpu.make_async_copy(dst_hbm.at[pl.ds(0, BB)],
                              dst_v.at[0], isem).wait()
        return 0

    lax.fori_loop(0, NB, _idrain, 0)
    pltpu.make_async_copy(src_hbm.at[pl.ds(0, EPT)], src_v, isem).wait()
    plsc.subcore_barrier()

    def _gather(j, b):
        pltpu.async_copy(xs_hbm.at[src_v.at[pl.ds(j * BB, BB)]],
                         rows_v.at[b], gsem[b])

    def _gwait(b):
        pltpu.make_async_copy(xs_hbm.at[src_v.at[pl.ds(0, BB)]],
                              rows_v.at[b], gsem[b]).wait()

    def _scatter(j, b):
        pltpu.async_copy(rows_v.at[b], tbl.at[dst_v.at[j]], ssem[b], add=True)

    def _swait(b):
        pltpu.make_async_copy(
            rows_v.at[b], tbl.at[dst_v.at[0]], ssem[b]).wait()

    for b in range(DEPTH):
        _gather(b, b)

    def _round(i, _):
        for b in range(DEPTH):
            j = DEPTH * i + b
            _gwait(b)
            _scatter(j, b)
            _swait(b)
            _gather(j + DEPTH, b)
        return 0

    lax.fori_loop(0, 30, _round, 0)
    for b in range(DEPTH):
        j = 120 + b
        _gwait(b)
        _scatter(j, b)
        _swait(b)
        if b == 0:
            _gather(124, 0)
    _gwait(0)
    _scatter(124, 0)
    _swait(0)

    plsc.subcore_barrier()
    pltpu.sync_copy(tbl.at[pl.ds(s * RPS, RPS)], zbuf_v)
    pltpu.sync_copy(zbuf_v, out_hbm.at[c, pl.ds(s * RPS, RPS)])


def _tc_edges_body(e_ref, src_ref, dst_ref):
    src_ref[...] = e_ref[0]
    dst_ref[...] = e_ref[1]


_tc_edges = pl.pallas_call(
    _tc_edges_body,
    out_shape=[
        jax.ShapeDtypeStruct((E,), jnp.int32),
        jax.ShapeDtypeStruct((E,), jnp.int32),
    ],
)


def _tc_matmul_body(x_ref, w1_ref, hw_ref):
    hw_ref[...] = jnp.dot(x_ref[...], w1_ref[...],
                          preferred_element_type=jnp.float32)


_tc_matmul = pl.pallas_call(
    _tc_matmul_body,
    grid=(N // BLK,),
    in_specs=[
        pl.BlockSpec((BLK, F_IN), lambda j: (j, 0)),
        pl.BlockSpec((F_IN, H), lambda j: (0, 0)),
    ],
    out_specs=pl.BlockSpec((BLK, H), lambda j: (j, 0)),
    out_shape=jax.ShapeDtypeStruct((N, H), jnp.float32),
)


def _tc_scale_body(deg_ref, hw_ref, norm_ref, xs1_ref):
    nrm = lax.rsqrt(1.0 + deg_ref[0] + deg_ref[1])
    norm_ref[...] = nrm
    xs1_ref[...] = hw_ref[...] * nrm


_tc_scale = pl.pallas_call(
    _tc_scale_body,
    grid=(N // BLK,),
    in_specs=[
        pl.BlockSpec((NC, BLK, 1), lambda j: (0, j, 0)),
        pl.BlockSpec((BLK, H), lambda j: (j, 0)),
    ],
    out_specs=[
        pl.BlockSpec((BLK, 1), lambda j: (j, 0)),
        pl.BlockSpec((BLK, H), lambda j: (j, 0)),
    ],
    out_shape=[
        jax.ShapeDtypeStruct((N, 1), jnp.float32),
        jax.ShapeDtypeStruct((NPAD, H), jnp.float32),
    ],
)


def _tc_mid_body(agg_ref, xs1_ref, norm_ref, b1_ref, xs2_ref):
    tot = agg_ref[0] + agg_ref[1] + xs1_ref[...]
    pre = tot * norm_ref[...] + b1_ref[...]
    xs2_ref[...] = jnp.maximum(pre, 0.0) * norm_ref[...]


_tc_mid = pl.pallas_call(
    _tc_mid_body,
    grid=(N // BLK,),
    in_specs=[
        pl.BlockSpec((NC, BLK, H), lambda j: (0, j, 0)),
        pl.BlockSpec((BLK, H), lambda j: (j, 0)),
        pl.BlockSpec((BLK, 1), lambda j: (j, 0)),
        pl.BlockSpec((1, H), lambda j: (0, 0)),
    ],
    out_specs=pl.BlockSpec((BLK, H), lambda j: (j, 0)),
    out_shape=jax.ShapeDtypeStruct((NPAD, H), jnp.float32),
)


def _tc_out_body(agg_ref, xs2_ref, norm_ref, w2_ref, b2_ref, out_ref):
    z = (agg_ref[0] + agg_ref[1] + xs2_ref[...]) * norm_ref[...]
    logits = jnp.dot(z, w2_ref[...], preferred_element_type=jnp.float32)
    logits = logits + b2_ref[...]
    m = jnp.max(logits, axis=1, keepdims=True)
    shifted = logits - m
    lse = jnp.log(jnp.sum(jnp.exp(shifted), axis=1, keepdims=True))
    out_ref[...] = shifted - lse


_tc_out = pl.pallas_call(
    _tc_out_body,
    grid=(N // BLK,),
    in_specs=[
        pl.BlockSpec((NC, BLK, H), lambda j: (0, j, 0)),
        pl.BlockSpec((BLK, H), lambda j: (j, 0)),
        pl.BlockSpec((BLK, 1), lambda j: (j, 0)),
        pl.BlockSpec((H, C), lambda j: (0, 0)),
        pl.BlockSpec((1, C), lambda j: (0, 0)),
    ],
    out_specs=pl.BlockSpec((BLK, C), lambda j: (j, 0)),
    out_shape=jax.ShapeDtypeStruct((N, C), jnp.float32),
)


def kernel(x, edge_index, W1, b1, W2, b2):
    ones1 = jnp.ones((BB, 1), jnp.float32)
    z1 = jnp.zeros((RPS, 1), jnp.float32)
    z2 = jnp.zeros((RPS, H), jnp.float32)

    src1, dst1 = _tc_edges(edge_index)
    degp = _sc_degree(dst1, ones1, z1)
    hw = _tc_matmul(x, W1)
    norm, xs1 = _tc_scale(degp, hw)
    a1 = _sc_aggregate(xs1, src1, dst1, z2)
    xs2 = _tc_mid(a1, xs1, norm, b1.reshape(1, H))
    a2 = _sc_aggregate(xs2, src1, dst1, z2)
    return _tc_out(a2, xs2, norm, W2, b2.reshape(1, C))


# speedup vs baseline: 63.1586x; 1.1045x over previous
import functools

import jax
import jax.numpy as jnp
from jax import lax
from jax.experimental import pallas as pl
from jax.experimental.pallas import tpu as pltpu
from jax.experimental.pallas import tpu_sc as plsc

N = 10000
E = 320000
F_IN = 128
H = 16
C = 100

NC = 2
NS = 16
NW = NC * NS
EPT = E // NW
BB = 80
NB = EPT // BB
NPAD = 10240
RPS = NPAD // NS
BLK = 2000
DEPTH = 4

_mesh = plsc.VectorSubcoreMesh(core_axis_name="c", subcore_axis_name="s")
_sc_params = pltpu.CompilerParams(use_tc_tiling_on_sc=False)


@functools.partial(
    pl.kernel,
    out_type=jax.ShapeDtypeStruct((NC, NPAD, 1), jnp.float32),
    mesh=_mesh,
    scratch_types=[
        pltpu.VMEM((NB, BB), jnp.int32),
        pltpu.VMEM((BB, 1), jnp.float32),
        pltpu.VMEM((RPS, 1), jnp.float32),
        pltpu.VMEM_SHARED((NPAD, 1), jnp.float32),
        pltpu.SemaphoreType.DMA,
        pltpu.SemaphoreType.DMA,
    ],
    compiler_params=_sc_params,
)
def _sc_degree(dst_hbm, ones_hbm, zeros_hbm, out_hbm,
               idx_v, ones_v, buf_v, tbl, sem, isem):
    c = lax.axis_index("c")
    s = lax.axis_index("s")
    wid = c * NS + s
    base = wid * EPT

    def _ifire(j, _):
        pltpu.async_copy(dst_hbm.at[pl.ds(base + j * BB, BB)],
                         idx_v.at[j], isem)
        return 0

    lax.fori_loop(0, NB, _ifire, 0)

    pltpu.sync_copy(ones_hbm, ones_v)
    pltpu.sync_copy(zeros_hbm, buf_v)
    pltpu.sync_copy(buf_v, tbl.at[pl.ds(s * RPS, RPS)])

    def _idrain(j, _):
        pltpu.make_async_copy(dst_hbm.at[pl.ds(0, BB)],
                              idx_v.at[0], isem).wait()
        return 0

    lax.fori_loop(0, NB, _idrain, 0)
    plsc.subcore_barrier()

    def _fire(j, _):
        pltpu.async_copy(ones_v, tbl.at[idx_v.at[j]], sem, add=True)
        return 0

    lax.fori_loop(0, NB, _fire, 0)

    def _drain(j, _):
        pltpu.make_async_copy(ones_v, tbl.at[idx_v.at[0]], sem).wait()
        return 0

    lax.fori_loop(0, NB, _drain, 0)

    plsc.subcore_barrier()
    pltpu.sync_copy(tbl.at[pl.ds(s * RPS, RPS)], buf_v)
    pltpu.sync_copy(buf_v, out_hbm.at[c, pl.ds(s * RPS, RPS)])


@functools.partial(
    pl.kernel,
    out_type=jax.ShapeDtypeStruct((NC, NPAD, H), jnp.float32),
    mesh=_mesh,
    scratch_types=(
        [
            pltpu.VMEM((EPT,), jnp.int32),
            pltpu.VMEM((NB, BB), jnp.int32),
            pltpu.VMEM((DEPTH, BB, H), jnp.float32),
            pltpu.VMEM((RPS, H), jnp.float32),
            pltpu.VMEM((RPS, H), jnp.float32),
            pltpu.VMEM_SHARED((NPAD, H), jnp.float32),
            pltpu.VMEM_SHARED((NPAD, H), jnp.float32),
        ]
        + [pltpu.SemaphoreType.DMA] * (2 * DEPTH + 2)
    ),
    compiler_params=_sc_params,
)
def _sc_aggregate(xs_hbm, src_hbm, dst_hbm, zeros_hbm, out_hbm,
                  src_v, dst_v, rows_v, zbuf_v, sbuf_v, xs_sh, tbl,
                  g0, g1, g2, g3, s0, s1, s2, s3, isem, stg):
    c = lax.axis_index("c")
    s = lax.axis_index("s")
    wid = c * NS + s
    base = wid * EPT
    gsem = (g0, g1, g2, g3)
    ssem = (s0, s1, s2, s3)

    pltpu.async_copy(src_hbm.at[pl.ds(base, EPT)], src_v, isem)
    stage = pltpu.async_copy(xs_hbm.at[pl.ds(s * RPS, RPS)], sbuf_v, stg)

    def _ifire(j, _):
        pltpu.async_copy(dst_hbm.at[pl.ds(base + j * BB, BB)],
                         dst_v.at[j], isem)
        return 0

    lax.fori_loop(0, NB, _ifire, 0)

    pltpu.sync_copy(zeros_hbm, zbuf_v)
    pltpu.sync_copy(zbuf_v, tbl.at[pl.ds(s * RPS, RPS)])
    stage.wait()
    pltpu.sync_copy(sbuf_v, xs_sh.at[pl.ds(s * RPS, RPS)])

    def _idrain(j, _):
        pltpu.make_async_copy(dst_hbm.at[pl.ds(0, BB)],
                              dst_v.at[0], isem).wait()
        return 0

    lax.fori_loop(0, NB, _idrain, 0)
    pltpu.make_async_copy(src_hbm.at[pl.ds(0, EPT)], src_v, isem).wait()
    plsc.subcore_barrier()

    def _gather(j, b):
        pltpu.async_copy(xs_sh.at[src_v.at[pl.ds(j * BB, BB)]],
                         rows_v.at[b], gsem[b])

    def _gwait(b):
        pltpu.make_async_copy(xs_sh.at[src_v.at[pl.ds(0, BB)]],
                              rows_v.at[b], gsem[b]).wait()

    def _scatter(j, b):
        pltpu.async_copy(rows_v.at[b], tbl.at[dst_v.at[j]], ssem[b], add=True)

    def _swait(b):
        pltpu.make_async_copy(
            rows_v.at[b], tbl.at[dst_v.at[0]], ssem[b]).wait()

    for b in range(DEPTH):
        _gather(b, b)

    def _round(i, _):
        for b in range(DEPTH):
            j = DEPTH * i + b
            _gwait(b)
            _scatter(j, b)
            _swait(b)
            _gather(j + DEPTH, b)
        return 0

    lax.fori_loop(0, 30, _round, 0)
    for b in range(DEPTH):
        j = 120 + b
        _gwait(b)
        _scatter(j, b)
        _swait(b)
        if b == 0:
            _gather(124, 0)
    _gwait(0)
    _scatter(124, 0)
    _swait(0)

    plsc.subcore_barrier()
    pltpu.sync_copy(tbl.at[pl.ds(s * RPS, RPS)], zbuf_v)
    pltpu.sync_copy(zbuf_v, out_hbm.at[c, pl.ds(s * RPS, RPS)])


def _tc_edges_body(e_ref, src_ref, dst_ref):
    src_ref[...] = e_ref[0]
    dst_ref[...] = e_ref[1]


_tc_edges = pl.pallas_call(
    _tc_edges_body,
    out_shape=[
        jax.ShapeDtypeStruct((E,), jnp.int32),
        jax.ShapeDtypeStruct((E,), jnp.int32),
    ],
)


def _tc_matmul_body(x_ref, w1_ref, hw_ref):
    hw_ref[...] = jnp.dot(x_ref[...], w1_ref[...],
                          preferred_element_type=jnp.float32)


_tc_matmul = pl.pallas_call(
    _tc_matmul_body,
    grid=(N // BLK,),
    in_specs=[
        pl.BlockSpec((BLK, F_IN), lambda j: (j, 0)),
        pl.BlockSpec((F_IN, H), lambda j: (0, 0)),
    ],
    out_specs=pl.BlockSpec((BLK, H), lambda j: (j, 0)),
    out_shape=jax.ShapeDtypeStruct((N, H), jnp.float32),
)


def _tc_scale_body(deg_ref, hw_ref, norm_ref, xs1_ref):
    nrm = lax.rsqrt(1.0 + deg_ref[0] + deg_ref[1])
    norm_ref[...] = nrm
    xs1_ref[...] = hw_ref[...] * nrm


_tc_scale = pl.pallas_call(
    _tc_scale_body,
    grid=(N // BLK,),
    in_specs=[
        pl.BlockSpec((NC, BLK, 1), lambda j: (0, j, 0)),
        pl.BlockSpec((BLK, H), lambda j: (j, 0)),
    ],
    out_specs=[
        pl.BlockSpec((BLK, 1), lambda j: (j, 0)),
        pl.BlockSpec((BLK, H), lambda j: (j, 0)),
    ],
    out_shape=[
        jax.ShapeDtypeStruct((N, 1), jnp.float32),
        jax.ShapeDtypeStruct((NPAD, H), jnp.float32),
    ],
)


def _tc_mid_body(agg_ref, xs1_ref, norm_ref, b1_ref, xs2_ref):
    tot = agg_ref[0] + agg_ref[1] + xs1_ref[...]
    pre = tot * norm_ref[...] + b1_ref[...]
    xs2_ref[...] = jnp.maximum(pre, 0.0) * norm_ref[...]


_tc_mid = pl.pallas_call(
    _tc_mid_body,
    grid=(N // BLK,),
    in_specs=[
        pl.BlockSpec((NC, BLK, H), lambda j: (0, j, 0)),
        pl.BlockSpec((BLK, H), lambda j: (j, 0)),
        pl.BlockSpec((BLK, 1), lambda j: (j, 0)),
        pl.BlockSpec((1, H), lambda j: (0, 0)),
    ],
    out_specs=pl.BlockSpec((BLK, H), lambda j: (j, 0)),
    out_shape=jax.ShapeDtypeStruct((NPAD, H), jnp.float32),
)


def _tc_out_body(agg_ref, xs2_ref, norm_ref, w2_ref, b2_ref, out_ref):
    z = (agg_ref[0] + agg_ref[1] + xs2_ref[...]) * norm_ref[...]
    logits = jnp.dot(z, w2_ref[...], preferred_element_type=jnp.float32)
    logits = logits + b2_ref[...]
    m = jnp.max(logits, axis=1, keepdims=True)
    shifted = logits - m
    lse = jnp.log(jnp.sum(jnp.exp(shifted), axis=1, keepdims=True))
    out_ref[...] = shifted - lse


_tc_out = pl.pallas_call(
    _tc_out_body,
    grid=(N // BLK,),
    in_specs=[
        pl.BlockSpec((NC, BLK, H), lambda j: (0, j, 0)),
        pl.BlockSpec((BLK, H), lambda j: (j, 0)),
        pl.BlockSpec((BLK, 1), lambda j: (j, 0)),
        pl.BlockSpec((H, C), lambda j: (0, 0)),
        pl.BlockSpec((1, C), lambda j: (0, 0)),
    ],
    out_specs=pl.BlockSpec((BLK, C), lambda j: (j, 0)),
    out_shape=jax.ShapeDtypeStruct((N, C), jnp.float32),
)


def kernel(x, edge_index, W1, b1, W2, b2):
    ones1 = jnp.ones((BB, 1), jnp.float32)
    z1 = jnp.zeros((RPS, 1), jnp.float32)
    z2 = jnp.zeros((RPS, H), jnp.float32)

    src1, dst1 = _tc_edges(edge_index)
    degp = _sc_degree(dst1, ones1, z1)
    hw = _tc_matmul(x, W1)
    norm, xs1 = _tc_scale(degp, hw)
    a1 = _sc_aggregate(xs1, src1, dst1, z2)
    xs2 = _tc_mid(a1, xs1, norm, b1.reshape(1, H))
    a2 = _sc_aggregate(xs2, src1, dst1, z2)
    return _tc_out(a2, xs2, norm, W2, b2.reshape(1, C))


# speedup vs baseline: 79.8121x; 1.2637x over previous
import functools

import jax
import jax.numpy as jnp
from jax import lax
from jax.experimental import pallas as pl
from jax.experimental.pallas import tpu as pltpu
from jax.experimental.pallas import tpu_sc as plsc

N = 10000
E = 320000
F_IN = 128
H = 16
C = 100

NC = 2
NS = 16
NW = NC * NS
EPT = E // NW
BB = 80
NB = EPT // BB
NPAD = 10240
RPS = NPAD // NS
BLK = 2000
DEPTH = 4

_mesh = plsc.VectorSubcoreMesh(core_axis_name="c", subcore_axis_name="s")
_sc_params = pltpu.CompilerParams(use_tc_tiling_on_sc=False)


@functools.partial(
    pl.kernel,
    out_type=jax.ShapeDtypeStruct((NC, NPAD, H), jnp.float32),
    mesh=_mesh,
    scratch_types=[
        pltpu.VMEM((NB, BB), jnp.int32),
        pltpu.VMEM((BB, H), jnp.float32),
        pltpu.VMEM((RPS, H), jnp.float32),
        pltpu.VMEM_SHARED((NPAD, H), jnp.float32),
        pltpu.SemaphoreType.DMA,
        pltpu.SemaphoreType.DMA,
    ],
    compiler_params=_sc_params,
)
def _sc_degree(dst_hbm, ones_hbm, zeros_hbm, out_hbm,
               idx_v, ones_v, buf_v, tbl, sem, isem):
    c = lax.axis_index("c")
    s = lax.axis_index("s")
    wid = c * NS + s
    base = wid * EPT

    def _ifire(j, _):
        pltpu.async_copy(dst_hbm.at[pl.ds(base + j * BB, BB)],
                         idx_v.at[j], isem)
        return 0

    lax.fori_loop(0, NB, _ifire, 0)

    pltpu.sync_copy(ones_hbm, ones_v)
    pltpu.sync_copy(zeros_hbm, buf_v)
    pltpu.sync_copy(buf_v, tbl.at[pl.ds(s * RPS, RPS)])

    def _idrain(j, _):
        pltpu.make_async_copy(dst_hbm.at[pl.ds(0, BB)],
                              idx_v.at[0], isem).wait()
        return 0

    lax.fori_loop(0, NB, _idrain, 0)
    plsc.subcore_barrier()

    def _fire(j, _):
        pltpu.async_copy(ones_v, tbl.at[idx_v.at[j]], sem, add=True)
        return 0

    lax.fori_loop(0, NB, _fire, 0)

    def _drain(j, _):
        pltpu.make_async_copy(ones_v, tbl.at[idx_v.at[0]], sem).wait()
        return 0

    lax.fori_loop(0, NB, _drain, 0)

    plsc.subcore_barrier()
    pltpu.sync_copy(tbl.at[pl.ds(s * RPS, RPS)], buf_v)
    pltpu.sync_copy(buf_v, out_hbm.at[c, pl.ds(s * RPS, RPS)])


@functools.partial(
    pl.kernel,
    out_type=jax.ShapeDtypeStruct((NC, NPAD, H), jnp.float32),
    mesh=_mesh,
    scratch_types=(
        [
            pltpu.VMEM((EPT,), jnp.int32),
            pltpu.VMEM((NB, BB), jnp.int32),
            pltpu.VMEM((DEPTH, BB, H), jnp.float32),
            pltpu.VMEM((RPS, H), jnp.float32),
            pltpu.VMEM((RPS, H), jnp.float32),
            pltpu.VMEM_SHARED((NPAD, H), jnp.float32),
            pltpu.VMEM_SHARED((NPAD, H), jnp.float32),
        ]
        + [pltpu.SemaphoreType.DMA] * (2 * DEPTH + 2)
    ),
    compiler_params=_sc_params,
)
def _sc_aggregate(xs_hbm, src_hbm, dst_hbm, zeros_hbm, out_hbm,
                  src_v, dst_v, rows_v, zbuf_v, sbuf_v, xs_sh, tbl,
                  g0, g1, g2, g3, s0, s1, s2, s3, isem, stg):
    c = lax.axis_index("c")
    s = lax.axis_index("s")
    wid = c * NS + s
    base = wid * EPT
    gsem = (g0, g1, g2, g3)
    ssem = (s0, s1, s2, s3)

    pltpu.async_copy(src_hbm.at[pl.ds(base, EPT)], src_v, isem)
    stage = pltpu.async_copy(xs_hbm.at[pl.ds(s * RPS, RPS)], sbuf_v, stg)

    def _ifire(j, _):
        pltpu.async_copy(dst_hbm.at[pl.ds(base + j * BB, BB)],
                         dst_v.at[j], isem)
        return 0

    lax.fori_loop(0, NB, _ifire, 0)

    pltpu.sync_copy(zeros_hbm, zbuf_v)
    pltpu.sync_copy(zbuf_v, tbl.at[pl.ds(s * RPS, RPS)])
    stage.wait()
    pltpu.sync_copy(sbuf_v, xs_sh.at[pl.ds(s * RPS, RPS)])

    def _idrain(j, _):
        pltpu.make_async_copy(dst_hbm.at[pl.ds(0, BB)],
                              dst_v.at[0], isem).wait()
        return 0

    lax.fori_loop(0, NB, _idrain, 0)
    pltpu.make_async_copy(src_hbm.at[pl.ds(0, EPT)], src_v, isem).wait()
    plsc.subcore_barrier()

    def _gather(j, b):
        pltpu.async_copy(xs_sh.at[src_v.at[pl.ds(j * BB, BB)]],
                         rows_v.at[b], gsem[b])

    def _gwait(b):
        pltpu.make_async_copy(xs_sh.at[src_v.at[pl.ds(0, BB)]],
                              rows_v.at[b], gsem[b]).wait()

    def _scatter(j, b):
        pltpu.async_copy(rows_v.at[b], tbl.at[dst_v.at[j]], ssem[b], add=True)

    def _swait(b):
        pltpu.make_async_copy(
            rows_v.at[b], tbl.at[dst_v.at[0]], ssem[b]).wait()

    for b in range(DEPTH):
        _gather(b, b)

    def _round(i, _):
        for b in range(DEPTH):
            j = DEPTH * i + b
            _gwait(b)
            _scatter(j, b)
            _swait(b)
            _gather(j + DEPTH, b)
        return 0

    lax.fori_loop(0, 30, _round, 0)
    for b in range(DEPTH):
        j = 120 + b
        _gwait(b)
        _scatter(j, b)
        _swait(b)
        if b == 0:
            _gather(124, 0)
    _gwait(0)
    _scatter(124, 0)
    _swait(0)

    plsc.subcore_barrier()
    pltpu.sync_copy(tbl.at[pl.ds(s * RPS, RPS)], zbuf_v)
    pltpu.sync_copy(zbuf_v, out_hbm.at[c, pl.ds(s * RPS, RPS)])


def _tc_edges_body(e_ref, src_ref, dst_ref):
    src_ref[...] = e_ref[0]
    dst_ref[...] = e_ref[1]


_tc_edges = pl.pallas_call(
    _tc_edges_body,
    out_shape=[
        jax.ShapeDtypeStruct((E,), jnp.int32),
        jax.ShapeDtypeStruct((E,), jnp.int32),
    ],
)


NF = NPAD * H // 128
FB = NF // 5
NFV = N * H // 128
FBV = NFV // 5


def _tc_matmul_body(x8_ref, w1_ref, hw_ref):
    w1 = w1_ref[...]
    blocks = [
        jnp.pad(w1, ((0, 0), (16 * k, 112 - 16 * k))) for k in range(8)
    ]
    w1big = jnp.concatenate(blocks, axis=0)
    hw_ref[pl.ds(0, NFV), :] = jnp.dot(x8_ref[...], w1big,
                                       preferred_element_type=jnp.float32)


_tc_matmul = pl.pallas_call(
    _tc_matmul_body,
    out_shape=jax.ShapeDtypeStruct((NF, 128), jnp.float32),
)


def _tc_scale_body(deg_ref, hw_ref, norm_ref, xs1_ref):
    nrm = lax.rsqrt(1.0 + deg_ref[0] + deg_ref[1])
    norm_ref[...] = nrm
    xs1_ref[...] = hw_ref[...] * nrm


_tc_scale = pl.pallas_call(
    _tc_scale_body,
    grid=(5,),
    in_specs=[
        pl.BlockSpec((NC, FB, 128), lambda j: (0, j, 0)),
        pl.BlockSpec((FB, 128), lambda j: (j, 0)),
    ],
    out_specs=[
        pl.BlockSpec((FB, 128), lambda j: (j, 0)),
        pl.BlockSpec((FB, 128), lambda j: (j, 0)),
    ],
    out_shape=[
        jax.ShapeDtypeStruct((NF, 128), jnp.float32),
        jax.ShapeDtypeStruct((NF, 128), jnp.float32),
    ],
)


def _tc_mid_body(agg_ref, xs1_ref, norm_ref, b1_ref, xs2_ref):
    nrm = norm_ref[...]
    tot = agg_ref[0] + agg_ref[1] + xs1_ref[...]
    pre = tot * nrm + b1_ref[...]
    xs2_ref[...] = jnp.maximum(pre, 0.0) * nrm


_tc_mid = pl.pallas_call(
    _tc_mid_body,
    grid=(5,),
    in_specs=[
        pl.BlockSpec((NC, FB, 128), lambda j: (0, j, 0)),
        pl.BlockSpec((FB, 128), lambda j: (j, 0)),
        pl.BlockSpec((FB, 128), lambda j: (j, 0)),
        pl.BlockSpec((1, 128), lambda j: (0, 0)),
    ],
    out_specs=pl.BlockSpec((FB, 128), lambda j: (j, 0)),
    out_shape=jax.ShapeDtypeStruct((NF, 128), jnp.float32),
)


def _tc_z_body(agg_ref, xs2_ref, norm_ref, z_ref):
    z_ref[...] = (agg_ref[0] + agg_ref[1] + xs2_ref[...]) * norm_ref[...]


_tc_z = pl.pallas_call(
    _tc_z_body,
    grid=(5,),
    in_specs=[
        pl.BlockSpec((NC, FB, 128), lambda j: (0, j, 0)),
        pl.BlockSpec((FB, 128), lambda j: (j, 0)),
        pl.BlockSpec((FB, 128), lambda j: (j, 0)),
    ],
    out_specs=pl.BlockSpec((FB, 128), lambda j: (j, 0)),
    out_shape=jax.ShapeDtypeStruct((NF, 128), jnp.float32),
)


def _tc_out_body(z_ref, w2_ref, b2_ref, out_ref):
    logits = jnp.dot(z_ref[...], w2_ref[...],
                     preferred_element_type=jnp.float32)
    logits = logits + b2_ref[...]
    m = jnp.max(logits, axis=1, keepdims=True)
    shifted = logits - m
    lse = jnp.log(jnp.sum(jnp.exp(shifted), axis=1, keepdims=True))
    out_ref[...] = shifted - lse


_tc_out = pl.pallas_call(
    _tc_out_body,
    grid=(N // BLK,),
    in_specs=[
        pl.BlockSpec((BLK, H), lambda j: (j, 0)),
        pl.BlockSpec((H, C), lambda j: (0, 0)),
        pl.BlockSpec((1, C), lambda j: (0, 0)),
    ],
    out_specs=pl.BlockSpec((BLK, C), lambda j: (j, 0)),
    out_shape=jax.ShapeDtypeStruct((N, C), jnp.float32),
)


def kernel(x, edge_index, W1, b1, W2, b2):
    ones2 = jnp.ones((BB, H), jnp.float32)
    z2 = jnp.zeros((RPS, H), jnp.float32)

    src1, dst1 = _tc_edges(edge_index)
    degp = _sc_degree(dst1, ones2, z2)
    hw = _tc_matmul(x.reshape(N // 8, 1024), W1)
    norm, xs1 = _tc_scale(degp.reshape(NC, NF, 128), hw)
    a1 = _sc_aggregate(xs1.reshape(NPAD, H), src1, dst1, z2)
    xs2 = _tc_mid(a1.reshape(NC, NF, 128), xs1, norm,
                  jnp.tile(b1, 8).reshape(1, 128))
    a2 = _sc_aggregate(xs2.reshape(NPAD, H), src1, dst1, z2)
    zf = _tc_z(a2.reshape(NC, NF, 128), xs2, norm)
    z16 = zf[0:NFV].reshape(N, H)
    return _tc_out(z16, W2, b2.reshape(1, C))


# speedup vs baseline: 83.5206x; 1.0465x over previous
import functools

import jax
import jax.numpy as jnp
from jax import lax
from jax.experimental import pallas as pl
from jax.experimental.pallas import tpu as pltpu
from jax.experimental.pallas import tpu_sc as plsc

N = 10000
E = 320000
F_IN = 128
H = 16
C = 100

NC = 2
NS = 16
NW = NC * NS
EPT = E // NW
BB = 80
NB = EPT // BB
NPAD = 10240
RPS = NPAD // NS
BLK = 2000
DEPTH = 4

_mesh = plsc.VectorSubcoreMesh(core_axis_name="c", subcore_axis_name="s")
_sc_params = pltpu.CompilerParams(use_tc_tiling_on_sc=False)


@functools.partial(
    pl.kernel,
    out_type=jax.ShapeDtypeStruct((NC, NPAD, H), jnp.float32),
    mesh=_mesh,
    scratch_types=[
        pltpu.VMEM((NB, BB), jnp.int32),
        pltpu.VMEM((BB,), jnp.float32),
        pltpu.VMEM((RPS,), jnp.float32),
        pltpu.VMEM((RPS, H), jnp.float32),
        pltpu.VMEM_SHARED((NPAD,), jnp.float32),
        pltpu.SemaphoreType.DMA,
        pltpu.SemaphoreType.DMA,
    ],
    compiler_params=_sc_params,
)
def _sc_degree(dst_hbm, ones_hbm, zeros_hbm, out_hbm,
               idx_v, ones_v, buf_v, rep_v, tbl, sem, isem):
    c = lax.axis_index("c")
    s = lax.axis_index("s")
    wid = c * NS + s
    base = wid * EPT

    def _ifire(j, _):
        pltpu.async_copy(dst_hbm.at[pl.ds(base + j * BB, BB)],
                         idx_v.at[j], isem)
        return 0

    lax.fori_loop(0, NB, _ifire, 0)

    pltpu.sync_copy(ones_hbm, ones_v)
    pltpu.sync_copy(zeros_hbm, buf_v)
    pltpu.sync_copy(buf_v, tbl.at[pl.ds(s * RPS, RPS)])

    def _idrain(j, _):
        pltpu.make_async_copy(dst_hbm.at[pl.ds(0, BB)],
                              idx_v.at[0], isem).wait()
        return 0

    lax.fori_loop(0, NB, _idrain, 0)
    plsc.subcore_barrier()

    def _fire(j, _):
        pltpu.async_copy(ones_v, tbl.at[idx_v.at[j]], sem, add=True)
        return 0

    lax.fori_loop(0, NB, _fire, 0)

    def _drain(j, _):
        pltpu.make_async_copy(ones_v, tbl.at[idx_v.at[0]], sem).wait()
        return 0

    lax.fori_loop(0, NB, _drain, 0)

    plsc.subcore_barrier()
    pltpu.sync_copy(tbl.at[pl.ds(s * RPS, RPS)], buf_v)

    def _rep(i, _):
        v = buf_v[pl.ds(i * 16, 16)]
        for k in range(16):
            rep_v[i * 16 + k] = jnp.full((H,), v[k], jnp.float32)
        return 0

    lax.fori_loop(0, RPS // 16, _rep, 0)
    pltpu.sync_copy(rep_v, out_hbm.at[c, pl.ds(s * RPS, RPS)])


@functools.partial(
    pl.kernel,
    out_type=jax.ShapeDtypeStruct((NC, NPAD, H), jnp.float32),
    mesh=_mesh,
    scratch_types=(
        [
            pltpu.VMEM((EPT,), jnp.int32),
            pltpu.VMEM((NB, BB), jnp.int32),
            pltpu.VMEM((DEPTH, BB, H), jnp.float32),
            pltpu.VMEM((RPS, H), jnp.float32),
            pltpu.VMEM((RPS, H), jnp.float32),
            pltpu.VMEM_SHARED((NPAD, H), jnp.float32),
            pltpu.VMEM_SHARED((NPAD, H), jnp.float32),
        ]
        + [pltpu.SemaphoreType.DMA] * (2 * DEPTH + 2)
    ),
    compiler_params=_sc_params,
)
def _sc_aggregate(xs_hbm, src_hbm, dst_hbm, zeros_hbm, out_hbm,
                  src_v, dst_v, rows_v, zbuf_v, sbuf_v, xs_sh, tbl,
                  g0, g1, g2, g3, s0, s1, s2, s3, isem, stg):
    c = lax.axis_index("c")
    s = lax.axis_index("s")
    wid = c * NS + s
    base = wid * EPT
    gsem = (g0, g1, g2, g3)
    ssem = (s0, s1, s2, s3)

    pltpu.async_copy(src_hbm.at[pl.ds(base, EPT)], src_v, isem)
    stage = pltpu.async_copy(xs_hbm.at[pl.ds(s * RPS, RPS)], sbuf_v, stg)

    def _ifire(j, _):
        pltpu.async_copy(dst_hbm.at[pl.ds(base + j * BB, BB)],
                         dst_v.at[j], isem)
        return 0

    lax.fori_loop(0, NB, _ifire, 0)

    pltpu.sync_copy(zeros_hbm, zbuf_v)
    pltpu.sync_copy(zbuf_v, tbl.at[pl.ds(s * RPS, RPS)])
    stage.wait()
    pltpu.sync_copy(sbuf_v, xs_sh.at[pl.ds(s * RPS, RPS)])

    def _idrain(j, _):
        pltpu.make_async_copy(dst_hbm.at[pl.ds(0, BB)],
                              dst_v.at[0], isem).wait()
        return 0

    lax.fori_loop(0, NB, _idrain, 0)
    pltpu.make_async_copy(src_hbm.at[pl.ds(0, EPT)], src_v, isem).wait()
    plsc.subcore_barrier()

    def _gather(j, b):
        pltpu.async_copy(xs_sh.at[src_v.at[pl.ds(j * BB, BB)]],
                         rows_v.at[b], gsem[b])

    def _gwait(b):
        pltpu.make_async_copy(xs_sh.at[src_v.at[pl.ds(0, BB)]],
                              rows_v.at[b], gsem[b]).wait()

    def _scatter(j, b):
        pltpu.async_copy(rows_v.at[b], tbl.at[dst_v.at[j]], ssem[b], add=True)

    def _swait(b):
        pltpu.make_async_copy(
            rows_v.at[b], tbl.at[dst_v.at[0]], ssem[b]).wait()

    for b in range(DEPTH):
        _gather(b, b)

    def _round(i, _):
        for b in range(DEPTH):
            j = DEPTH * i + b
            _gwait(b)
            _scatter(j, b)
            _swait(b)
            _gather(j + DEPTH, b)
        return 0

    lax.fori_loop(0, 30, _round, 0)
    for b in range(DEPTH):
        j = 120 + b
        _gwait(b)
        _scatter(j, b)
        _swait(b)
        if b == 0:
            _gather(124, 0)
    _gwait(0)
    _scatter(124, 0)
    _swait(0)

    plsc.subcore_barrier()
    pltpu.sync_copy(tbl.at[pl.ds(s * RPS, RPS)], zbuf_v)
    pltpu.sync_copy(zbuf_v, out_hbm.at[c, pl.ds(s * RPS, RPS)])


def _tc_edges_body(e_ref, src_ref, dst_ref):
    src_ref[...] = e_ref[0]
    dst_ref[...] = e_ref[1]


_tc_edges = pl.pallas_call(
    _tc_edges_body,
    out_shape=[
        jax.ShapeDtypeStruct((E,), jnp.int32),
        jax.ShapeDtypeStruct((E,), jnp.int32),
    ],
)


NF = NPAD * H // 128
FB = NF // 5
NFV = N * H // 128
FBV = NFV // 5


def _tc_matmul_body(x8_ref, w1_ref, hw_ref):
    w1 = w1_ref[...]
    blocks = [
        jnp.pad(w1, ((0, 0), (16 * k, 112 - 16 * k))) for k in range(8)
    ]
    w1big = jnp.concatenate(blocks, axis=0)
    hw_ref[pl.ds(0, NFV), :] = jnp.dot(x8_ref[...], w1big,
                                       preferred_element_type=jnp.float32)


_tc_matmul = pl.pallas_call(
    _tc_matmul_body,
    out_shape=jax.ShapeDtypeStruct((NF, 128), jnp.float32),
)


def _tc_scale_body(deg_ref, hw_ref, norm_ref, xs1_ref):
    nrm = lax.rsqrt(1.0 + deg_ref[0] + deg_ref[1])
    norm_ref[...] = nrm
    xs1_ref[...] = hw_ref[...] * nrm


_tc_scale = pl.pallas_call(
    _tc_scale_body,
    grid=(5,),
    in_specs=[
        pl.BlockSpec((NC, FB, 128), lambda j: (0, j, 0)),
        pl.BlockSpec((FB, 128), lambda j: (j, 0)),
    ],
    out_specs=[
        pl.BlockSpec((FB, 128), lambda j: (j, 0)),
        pl.BlockSpec((FB, 128), lambda j: (j, 0)),
    ],
    out_shape=[
        jax.ShapeDtypeStruct((NF, 128), jnp.float32),
        jax.ShapeDtypeStruct((NF, 128), jnp.float32),
    ],
)


def _tc_mid_body(agg_ref, xs1_ref, norm_ref, b1_ref, xs2_ref):
    nrm = norm_ref[...]
    tot = agg_ref[0] + agg_ref[1] + xs1_ref[...]
    pre = tot * nrm + b1_ref[...]
    xs2_ref[...] = jnp.maximum(pre, 0.0) * nrm


_tc_mid = pl.pallas_call(
    _tc_mid_body,
    grid=(5,),
    in_specs=[
        pl.BlockSpec((NC, FB, 128), lambda j: (0, j, 0)),
        pl.BlockSpec((FB, 128), lambda j: (j, 0)),
        pl.BlockSpec((FB, 128), lambda j: (j, 0)),
        pl.BlockSpec((1, 128), lambda j: (0, 0)),
    ],
    out_specs=pl.BlockSpec((FB, 128), lambda j: (j, 0)),
    out_shape=jax.ShapeDtypeStruct((NF, 128), jnp.float32),
)


def _tc_z_body(agg_ref, xs2_ref, norm_ref, z_ref):
    z_ref[...] = (agg_ref[0] + agg_ref[1] + xs2_ref[...]) * norm_ref[...]


_tc_z = pl.pallas_call(
    _tc_z_body,
    grid=(5,),
    in_specs=[
        pl.BlockSpec((NC, FB, 128), lambda j: (0, j, 0)),
        pl.BlockSpec((FB, 128), lambda j: (j, 0)),
        pl.BlockSpec((FB, 128), lambda j: (j, 0)),
    ],
    out_specs=pl.BlockSpec((FB, 128), lambda j: (j, 0)),
    out_shape=jax.ShapeDtypeStruct((NF, 128), jnp.float32),
)


def _tc_out_body(z_ref, w2_ref, b2_ref, out_ref):
    logits = jnp.dot(z_ref[...], w2_ref[...],
                     preferred_element_type=jnp.float32)
    logits = logits + b2_ref[...]
    m = jnp.max(logits, axis=1, keepdims=True)
    shifted = logits - m
    lse = jnp.log(jnp.sum(jnp.exp(shifted), axis=1, keepdims=True))
    out_ref[...] = shifted - lse


_tc_out = pl.pallas_call(
    _tc_out_body,
    grid=(N // BLK,),
    in_specs=[
        pl.BlockSpec((BLK, H), lambda j: (j, 0)),
        pl.BlockSpec((H, C), lambda j: (0, 0)),
        pl.BlockSpec((1, C), lambda j: (0, 0)),
    ],
    out_specs=pl.BlockSpec((BLK, C), lambda j: (j, 0)),
    out_shape=jax.ShapeDtypeStruct((N, C), jnp.float32),
)


def kernel(x, edge_index, W1, b1, W2, b2):
    ones1 = jnp.ones((BB,), jnp.float32)
    z1 = jnp.zeros((RPS,), jnp.float32)
    z2 = jnp.zeros((RPS, H), jnp.float32)

    src1, dst1 = _tc_edges(edge_index)
    degp = _sc_degree(dst1, ones1, z1)
    hw = _tc_matmul(x.reshape(N // 8, 1024), W1)
    norm, xs1 = _tc_scale(degp.reshape(NC, NF, 128), hw)
    a1 = _sc_aggregate(xs1.reshape(NPAD, H), src1, dst1, z2)
    xs2 = _tc_mid(a1.reshape(NC, NF, 128), xs1, norm,
                  jnp.tile(b1, 8).reshape(1, 128))
    a2 = _sc_aggregate(xs2.reshape(NPAD, H), src1, dst1, z2)
    zf = _tc_z(a2.reshape(NC, NF, 128), xs2, norm)
    z16 = zf[0:NFV].reshape(N, H)
    return _tc_out(z16, W2, b2.reshape(1, C))


# speedup vs baseline: 87.6909x; 1.0499x over previous
import functools

import jax
import jax.numpy as jnp
from jax import lax
from jax.experimental import pallas as pl
from jax.experimental.pallas import tpu as pltpu
from jax.experimental.pallas import tpu_sc as plsc

N = 10000
E = 320000
F_IN = 128
H = 16
C = 100

NC = 2
NS = 16
NW = NC * NS
EPT = E // NW
BB = 80
NB = EPT // BB
NPAD = 10240
RPS = NPAD // NS
BLK = 2000
DEPTH = 4

_mesh = plsc.VectorSubcoreMesh(core_axis_name="c", subcore_axis_name="s")
_sc_params = pltpu.CompilerParams(use_tc_tiling_on_sc=False)


@functools.partial(
    pl.kernel,
    out_type=jax.ShapeDtypeStruct((NC, NPAD, H), jnp.float32),
    mesh=_mesh,
    scratch_types=[
        pltpu.VMEM((NB, BB), jnp.int32),
        pltpu.VMEM((BB,), jnp.float32),
        pltpu.VMEM((RPS,), jnp.float32),
        pltpu.VMEM((RPS, H), jnp.float32),
        pltpu.VMEM_SHARED((NPAD,), jnp.float32),
        pltpu.SemaphoreType.DMA,
        pltpu.SemaphoreType.DMA,
    ],
    compiler_params=_sc_params,
)
def _sc_degree(dst_hbm, ones_hbm, zeros_hbm, out_hbm,
               idx_v, ones_v, buf_v, rep_v, tbl, sem, isem):
    c = lax.axis_index("c")
    s = lax.axis_index("s")
    wid = c * NS + s
    base = wid * EPT

    def _ifire(j, _):
        pltpu.async_copy(dst_hbm.at[pl.ds(base + j * BB, BB)],
                         idx_v.at[j], isem)
        return 0

    lax.fori_loop(0, NB, _ifire, 0)

    pltpu.sync_copy(ones_hbm, ones_v)
    pltpu.sync_copy(zeros_hbm, buf_v)
    pltpu.sync_copy(buf_v, tbl.at[pl.ds(s * RPS, RPS)])

    def _idrain(j, _):
        pltpu.make_async_copy(dst_hbm.at[pl.ds(0, BB)],
                              idx_v.at[0], isem).wait()
        return 0

    lax.fori_loop(0, NB, _idrain, 0)
    plsc.subcore_barrier()

    def _fire(j, _):
        pltpu.async_copy(ones_v, tbl.at[idx_v.at[j]], sem, add=True)
        return 0

    lax.fori_loop(0, NB, _fire, 0)

    def _drain(j, _):
        pltpu.make_async_copy(ones_v, tbl.at[idx_v.at[0]], sem).wait()
        return 0

    lax.fori_loop(0, NB, _drain, 0)

    plsc.subcore_barrier()
    pltpu.sync_copy(tbl.at[pl.ds(s * RPS, RPS)], buf_v)

    def _rep(i, _):
        v = buf_v[pl.ds(i * 16, 16)]
        for k in range(16):
            rep_v[i * 16 + k] = jnp.full((H,), v[k], jnp.float32)
        return 0

    lax.fori_loop(0, RPS // 16, _rep, 0)
    pltpu.sync_copy(rep_v, out_hbm.at[c, pl.ds(s * RPS, RPS)])


@functools.partial(
    pl.kernel,
    out_type=jax.ShapeDtypeStruct((NC, NPAD, H), jnp.float32),
    mesh=_mesh,
    scratch_types=(
        [
            pltpu.VMEM((EPT,), jnp.int32),
            pltpu.VMEM((NB, BB), jnp.int32),
            pltpu.VMEM((DEPTH, BB, H), jnp.float32),
            pltpu.VMEM((RPS, H), jnp.float32),
            pltpu.VMEM((RPS, H), jnp.float32),
            pltpu.VMEM_SHARED((NPAD, H), jnp.float32),
            pltpu.VMEM_SHARED((NPAD, H), jnp.float32),
        ]
        + [pltpu.SemaphoreType.DMA] * (2 * DEPTH + 2)
    ),
    compiler_params=_sc_params,
)
def _sc_aggregate(xs_hbm, src_hbm, dst_hbm, zeros_hbm, out_hbm,
                  src_v, dst_v, rows_v, zbuf_v, sbuf_v, xs_sh, tbl,
                  g0, g1, g2, g3, s0, s1, s2, s3, isem, stg):
    c = lax.axis_index("c")
    s = lax.axis_index("s")
    wid = c * NS + s
    base = wid * EPT
    gsem = (g0, g1, g2, g3)
    ssem = (s0, s1, s2, s3)

    pltpu.async_copy(src_hbm.at[pl.ds(base, EPT)], src_v, isem)
    stage = pltpu.async_copy(xs_hbm.at[pl.ds(s * RPS, RPS)], sbuf_v, stg)

    def _ifire(j, _):
        pltpu.async_copy(dst_hbm.at[pl.ds(base + j * BB, BB)],
                         dst_v.at[j], isem)
        return 0

    lax.fori_loop(0, NB, _ifire, 0)

    pltpu.sync_copy(zeros_hbm, zbuf_v)
    pltpu.sync_copy(zbuf_v, tbl.at[pl.ds(s * RPS, RPS)])
    stage.wait()
    pltpu.sync_copy(sbuf_v, xs_sh.at[pl.ds(s * RPS, RPS)])

    def _idrain(j, _):
        pltpu.make_async_copy(dst_hbm.at[pl.ds(0, BB)],
                              dst_v.at[0], isem).wait()
        return 0

    lax.fori_loop(0, NB, _idrain, 0)
    pltpu.make_async_copy(src_hbm.at[pl.ds(0, EPT)], src_v, isem).wait()
    plsc.subcore_barrier()

    def _gather(j, b):
        pltpu.async_copy(xs_sh.at[src_v.at[pl.ds(j * BB, BB)]],
                         rows_v.at[b], gsem[b])

    def _gwait(b):
        pltpu.make_async_copy(xs_sh.at[src_v.at[pl.ds(0, BB)]],
                              rows_v.at[b], gsem[b]).wait()

    def _scatter(j, b):
        pltpu.async_copy(rows_v.at[b], tbl.at[dst_v.at[j]], ssem[b], add=True)

    def _swait(b):
        pltpu.make_async_copy(
            rows_v.at[b], tbl.at[dst_v.at[0]], ssem[b]).wait()

    for b in range(DEPTH):
        _gather(b, b)

    def _round(i, _):
        for b in range(DEPTH):
            j = DEPTH * i + b
            _gwait(b)
            _scatter(j, b)
            _swait(b)
            _gather(j + DEPTH, b)
        return 0

    lax.fori_loop(0, 30, _round, 0)
    for b in range(DEPTH):
        j = 120 + b
        _gwait(b)
        _scatter(j, b)
        _swait(b)
        if b == 0:
            _gather(124, 0)
    _gwait(0)
    _scatter(124, 0)
    _swait(0)

    plsc.subcore_barrier()
    pltpu.sync_copy(tbl.at[pl.ds(s * RPS, RPS)], zbuf_v)
    pltpu.sync_copy(zbuf_v, out_hbm.at[c, pl.ds(s * RPS, RPS)])


def _tc_edges_body(e_ref, src_ref, dst_ref):
    src_ref[...] = e_ref[0]
    dst_ref[...] = e_ref[1]


_tc_edges = pl.pallas_call(
    _tc_edges_body,
    out_shape=[
        jax.ShapeDtypeStruct((E,), jnp.int32),
        jax.ShapeDtypeStruct((E,), jnp.int32),
    ],
)


NF = NPAD * H // 128
FB = NF // 5
NFV = N * H // 128
FBV = NFV // 5


def _tc_matmul_body(x8_ref, w1_ref, hw_ref):
    w1 = w1_ref[...]
    blocks = [
        jnp.pad(w1, ((0, 0), (16 * k, 112 - 16 * k))) for k in range(8)
    ]
    w1big = jnp.concatenate(blocks, axis=0)
    hw_ref[pl.ds(0, NFV), :] = jnp.dot(x8_ref[...], w1big,
                                       preferred_element_type=jnp.float32)


_tc_matmul = pl.pallas_call(
    _tc_matmul_body,
    out_shape=jax.ShapeDtypeStruct((NF, 128), jnp.float32),
)


def _tc_scale_body(deg_ref, hw_ref, norm_ref, xs1_ref):
    nrm = lax.rsqrt(1.0 + deg_ref[0] + deg_ref[1])
    norm_ref[...] = nrm
    xs1_ref[...] = hw_ref[...] * nrm


_tc_scale = pl.pallas_call(
    _tc_scale_body,
    grid=(5,),
    in_specs=[
        pl.BlockSpec((NC, FB, 128), lambda j: (0, j, 0)),
        pl.BlockSpec((FB, 128), lambda j: (j, 0)),
    ],
    out_specs=[
        pl.BlockSpec((FB, 128), lambda j: (j, 0)),
        pl.BlockSpec((FB, 128), lambda j: (j, 0)),
    ],
    out_shape=[
        jax.ShapeDtypeStruct((NF, 128), jnp.float32),
        jax.ShapeDtypeStruct((NF, 128), jnp.float32),
    ],
)


def _tc_mid_body(agg_ref, xs1_ref, norm_ref, b1_ref, xs2_ref):
    nrm = norm_ref[...]
    tot = agg_ref[0] + agg_ref[1] + xs1_ref[...]
    pre = tot * nrm + b1_ref[...]
    xs2_ref[...] = jnp.maximum(pre, 0.0) * nrm


_tc_mid = pl.pallas_call(
    _tc_mid_body,
    grid=(5,),
    in_specs=[
        pl.BlockSpec((NC, FB, 128), lambda j: (0, j, 0)),
        pl.BlockSpec((FB, 128), lambda j: (j, 0)),
        pl.BlockSpec((FB, 128), lambda j: (j, 0)),
        pl.BlockSpec((1, 128), lambda j: (0, 0)),
    ],
    out_specs=pl.BlockSpec((FB, 128), lambda j: (j, 0)),
    out_shape=jax.ShapeDtypeStruct((NF, 128), jnp.float32),
)


def _tc_z_body(agg_ref, xs2_ref, norm_ref, z_ref):
    z_ref[...] = (agg_ref[0] + agg_ref[1] + xs2_ref[...]) * norm_ref[...]


_tc_z = pl.pallas_call(
    _tc_z_body,
    grid=(5,),
    in_specs=[
        pl.BlockSpec((NC, FB, 128), lambda j: (0, j, 0)),
        pl.BlockSpec((FB, 128), lambda j: (j, 0)),
        pl.BlockSpec((FB, 128), lambda j: (j, 0)),
    ],
    out_specs=pl.BlockSpec((FB, 128), lambda j: (j, 0)),
    out_shape=jax.ShapeDtypeStruct((NF, 128), jnp.float32),
)


def _tc_out_body(z_ref, w2_ref, b2_ref, out_ref):
    logits = jnp.dot(z_ref[...], w2_ref[...],
                     preferred_element_type=jnp.float32)
    logits = logits + b2_ref[...]
    m = jnp.max(logits, axis=1, keepdims=True)
    shifted = logits - m
    lse = jnp.log(jnp.sum(jnp.exp(shifted), axis=1, keepdims=True))
    out_ref[...] = (shifted - lse).T


_tc_out = pl.pallas_call(
    _tc_out_body,
    out_shape=jax.ShapeDtypeStruct((C, N), jnp.float32),
)


def kernel(x, edge_index, W1, b1, W2, b2):
    ones1 = jnp.ones((BB,), jnp.float32)
    z1 = jnp.zeros((RPS,), jnp.float32)
    z2 = jnp.zeros((RPS, H), jnp.float32)

    src1, dst1 = _tc_edges(edge_index)
    degp = _sc_degree(dst1, ones1, z1)
    hw = _tc_matmul(x.reshape(N // 8, 1024), W1)
    norm, xs1 = _tc_scale(degp.reshape(NC, NF, 128), hw)
    a1 = _sc_aggregate(xs1.reshape(NPAD, H), src1, dst1, z2)
    xs2 = _tc_mid(a1.reshape(NC, NF, 128), xs1, norm,
                  jnp.tile(b1, 8).reshape(1, 128))
    a2 = _sc_aggregate(xs2.reshape(NPAD, H), src1, dst1, z2)
    zf = _tc_z(a2.reshape(NC, NF, 128), xs2, norm)
    z16 = zf[0:NFV].reshape(N, H)
    return _tc_out(z16, W2, b2.reshape(1, C)).T


# speedup vs baseline: 89.2564x; 1.0179x over previous
import functools

import jax
import jax.numpy as jnp
from jax import lax
from jax.experimental import pallas as pl
from jax.experimental.pallas import tpu as pltpu
from jax.experimental.pallas import tpu_sc as plsc

N = 10000
E = 320000
F_IN = 128
H = 16
C = 100

NC = 2
NS = 16
NW = NC * NS
BB = 128
NB = 79
EPT = NB * BB
EPAD = NW * EPT - E
PADN = 10200
NPAD = 10240
RPS = NPAD // NS
BLK = 2000
DEPTH = 4

_mesh = plsc.VectorSubcoreMesh(core_axis_name="c", subcore_axis_name="s")
_sc_params = pltpu.CompilerParams(use_tc_tiling_on_sc=False)


@functools.partial(
    pl.kernel,
    out_type=jax.ShapeDtypeStruct((NC, NPAD, H), jnp.float32),
    mesh=_mesh,
    scratch_types=[
        pltpu.VMEM((NB, BB), jnp.int32),
        pltpu.VMEM((BB,), jnp.float32),
        pltpu.VMEM((RPS,), jnp.float32),
        pltpu.VMEM((RPS, H), jnp.float32),
        pltpu.VMEM_SHARED((NPAD,), jnp.float32),
        pltpu.SemaphoreType.DMA,
        pltpu.SemaphoreType.DMA,
    ],
    compiler_params=_sc_params,
)
def _sc_degree(dst_hbm, ones_hbm, zeros_hbm, out_hbm,
               idx_v, ones_v, buf_v, rep_v, tbl, sem, isem):
    c = lax.axis_index("c")
    s = lax.axis_index("s")
    wid = c * NS + s

    idx = pltpu.async_copy(dst_hbm.at[wid], idx_v, isem)
    pltpu.sync_copy(ones_hbm, ones_v)
    pltpu.sync_copy(zeros_hbm, buf_v)
    pltpu.sync_copy(buf_v, tbl.at[pl.ds(s * RPS, RPS)])
    idx.wait()
    plsc.subcore_barrier()

    def _fire(j, _):
        pltpu.async_copy(ones_v, tbl.at[idx_v.at[j]], sem, add=True)
        return 0

    lax.fori_loop(0, NB, _fire, 0)

    def _drain(j, _):
        pltpu.make_async_copy(ones_v, tbl.at[idx_v.at[0]], sem).wait()
        return 0

    lax.fori_loop(0, NB, _drain, 0)

    plsc.subcore_barrier()
    pltpu.sync_copy(tbl.at[pl.ds(s * RPS, RPS)], buf_v)

    def _rep(i, _):
        v = buf_v[pl.ds(i * 16, 16)]
        for k in range(16):
            rep_v[i * 16 + k] = jnp.full((H,), v[k], jnp.float32)
        return 0

    lax.fori_loop(0, RPS // 16, _rep, 0)
    pltpu.sync_copy(rep_v, out_hbm.at[c, pl.ds(s * RPS, RPS)])


@functools.partial(
    pl.kernel,
    out_type=jax.ShapeDtypeStruct((NC, NPAD, H), jnp.float32),
    mesh=_mesh,
    scratch_types=(
        [
            pltpu.VMEM((NB, BB), jnp.int32),
            pltpu.VMEM((NB, BB), jnp.int32),
            pltpu.VMEM((DEPTH, BB, H), jnp.float32),
            pltpu.VMEM((RPS, H), jnp.float32),
            pltpu.VMEM((RPS, H), jnp.float32),
            pltpu.VMEM_SHARED((NPAD, H), jnp.float32),
            pltpu.VMEM_SHARED((NPAD, H), jnp.float32),
        ]
        + [pltpu.SemaphoreType.DMA] * (2 * DEPTH + 2)
    ),
    compiler_params=_sc_params,
)
def _sc_aggregate(xs_hbm, src_hbm, dst_hbm, zeros_hbm, out_hbm,
                  src_v, dst_v, rows_v, zbuf_v, sbuf_v, xs_sh, tbl,
                  g0, g1, g2, g3, s0, s1, s2, s3, isem, stg):
    c = lax.axis_index("c")
    s = lax.axis_index("s")
    wid = c * NS + s
    gsem = (g0, g1, g2, g3)
    ssem = (s0, s1, s2, s3)

    pltpu.async_copy(src_hbm.at[wid], src_v, isem)
    stage = pltpu.async_copy(xs_hbm.at[pl.ds(s * RPS, RPS)], sbuf_v, stg)
    pltpu.async_copy(dst_hbm.at[wid], dst_v, isem)

    pltpu.sync_copy(zeros_hbm, zbuf_v)
    pltpu.sync_copy(zbuf_v, tbl.at[pl.ds(s * RPS, RPS)])
    stage.wait()
    pltpu.sync_copy(sbuf_v, xs_sh.at[pl.ds(s * RPS, RPS)])

    pltpu.make_async_copy(src_hbm.at[wid], src_v, isem).wait()
    pltpu.make_async_copy(dst_hbm.at[wid], dst_v, isem).wait()
    plsc.subcore_barrier()

    def _gather(j, b):
        pltpu.async_copy(xs_sh.at[src_v.at[j]], rows_v.at[b], gsem[b])

    def _gwait(b):
        pltpu.make_async_copy(xs_sh.at[src_v.at[0]],
                              rows_v.at[b], gsem[b]).wait()

    def _scatter(j, b):
        pltpu.async_copy(rows_v.at[b], tbl.at[dst_v.at[j]], ssem[b], add=True)

    def _swait(b):
        pltpu.make_async_copy(
            rows_v.at[b], tbl.at[dst_v.at[0]], ssem[b]).wait()

    for b in range(DEPTH):
        _gather(b, b)

    def _round(i, _):
        for b in range(DEPTH):
            j = DEPTH * i + b
            _gwait(b)
            _scatter(j, b)
            _swait(b)
            _gather(j + DEPTH, b)
        return 0

    lax.fori_loop(0, 18, _round, 0)
    for b in range(DEPTH):
        j = 72 + b
        _gwait(b)
        _scatter(j, b)
        _swait(b)
        if j + DEPTH < NB:
            _gather(j + DEPTH, b)
    for b in range(3):
        _gwait(b)
        _scatter(76 + b, b)
        _swait(b)

    plsc.subcore_barrier()
    pltpu.sync_copy(tbl.at[pl.ds(s * RPS, RPS)], zbuf_v)
    pltpu.sync_copy(zbuf_v, out_hbm.at[c, pl.ds(s * RPS, RPS)])


def _tc_edges_body(e_ref, src_ref, dst_ref):
    src_ref[...] = e_ref[0]
    dst_ref[...] = e_ref[1]


_tc_edges = pl.pallas_call(
    _tc_edges_body,
    out_shape=[
        jax.ShapeDtypeStruct((E,), jnp.int32),
        jax.ShapeDtypeStruct((E,), jnp.int32),
    ],
)


NF = NPAD * H // 128
FB = NF // 5
NFV = N * H // 128
FBV = NFV // 5


def _tc_matmul_body(x8_ref, w1_ref, hw_ref):
    w1 = w1_ref[...]
    blocks = [
        jnp.pad(w1, ((0, 0), (16 * k, 112 - 16 * k))) for k in range(8)
    ]
    w1big = jnp.concatenate(blocks, axis=0)
    hw_ref[pl.ds(0, NFV), :] = jnp.dot(x8_ref[...], w1big,
                                       preferred_element_type=jnp.float32)


_tc_matmul = pl.pallas_call(
    _tc_matmul_body,
    out_shape=jax.ShapeDtypeStruct((NF, 128), jnp.float32),
)


def _tc_scale_body(deg_ref, hw_ref, norm_ref, xs1_ref):
    nrm = lax.rsqrt(1.0 + deg_ref[0] + deg_ref[1])
    norm_ref[...] = nrm
    xs1_ref[...] = hw_ref[...] * nrm


_tc_scale = pl.pallas_call(
    _tc_scale_body,
    grid=(5,),
    in_specs=[
        pl.BlockSpec((NC, FB, 128), lambda j: (0, j, 0)),
        pl.BlockSpec((FB, 128), lambda j: (j, 0)),
    ],
    out_specs=[
        pl.BlockSpec((FB, 128), lambda j: (j, 0)),
        pl.BlockSpec((FB, 128), lambda j: (j, 0)),
    ],
    out_shape=[
        jax.ShapeDtypeStruct((NF, 128), jnp.float32),
        jax.ShapeDtypeStruct((NF, 128), jnp.float32),
    ],
)


def _tc_mid_body(agg_ref, xs1_ref, norm_ref, b1_ref, xs2_ref):
    nrm = norm_ref[...]
    tot = agg_ref[0] + agg_ref[1] + xs1_ref[...]
    pre = tot * nrm + b1_ref[...]
    xs2_ref[...] = jnp.maximum(pre, 0.0) * nrm


_tc_mid = pl.pallas_call(
    _tc_mid_body,
    grid=(5,),
    in_specs=[
        pl.BlockSpec((NC, FB, 128), lambda j: (0, j, 0)),
        pl.BlockSpec((FB, 128), lambda j: (j, 0)),
        pl.BlockSpec((FB, 128), lambda j: (j, 0)),
        pl.BlockSpec((1, 128), lambda j: (0, 0)),
    ],
    out_specs=pl.BlockSpec((FB, 128), lambda j: (j, 0)),
    out_shape=jax.ShapeDtypeStruct((NF, 128), jnp.float32),
)


def _tc_z_body(agg_ref, xs2_ref, norm_ref, z_ref):
    z_ref[...] = (agg_ref[0] + agg_ref[1] + xs2_ref[...]) * norm_ref[...]


_tc_z = pl.pallas_call(
    _tc_z_body,
    grid=(5,),
    in_specs=[
        pl.BlockSpec((NC, FB, 128), lambda j: (0, j, 0)),
        pl.BlockSpec((FB, 128), lambda j: (j, 0)),
        pl.BlockSpec((FB, 128), lambda j: (j, 0)),
    ],
    out_specs=pl.BlockSpec((FB, 128), lambda j: (j, 0)),
    out_shape=jax.ShapeDtypeStruct((NF, 128), jnp.float32),
)


def _tc_out_body(z_ref, w2_ref, b2_ref, out_ref):
    logits = jnp.dot(z_ref[...], w2_ref[...],
                     preferred_element_type=jnp.float32)
    logits = logits + b2_ref[...]
    m = jnp.max(logits, axis=1, keepdims=True)
    shifted = logits - m
    lse = jnp.log(jnp.sum(jnp.exp(shifted), axis=1, keepdims=True))
    out_ref[...] = (shifted - lse).T


_tc_out = pl.pallas_call(
    _tc_out_body,
    out_shape=jax.ShapeDtypeStruct((C, N), jnp.float32),
)


def kernel(x, edge_index, W1, b1, W2, b2):
    ones1 = jnp.ones((BB,), jnp.float32)
    z1 = jnp.zeros((RPS,), jnp.float32)
    z2 = jnp.zeros((RPS, H), jnp.float32)

    src1, dst1 = _tc_edges(edge_index)
    pad = jnp.full((EPAD,), PADN, jnp.int32)
    src3 = jnp.concatenate([src1, pad]).reshape(NW, NB, BB)
    dst3 = jnp.concatenate([dst1, pad]).reshape(NW, NB, BB)
    degp = _sc_degree(dst3, ones1, z1)
    hw = _tc_matmul(x.reshape(N // 8, 1024), W1)
    norm, xs1 = _tc_scale(degp.reshape(NC, NF, 128), hw)
    a1 = _sc_aggregate(xs1.reshape(NPAD, H), src3, dst3, z2)
    xs2 = _tc_mid(a1.reshape(NC, NF, 128), xs1, norm,
                  jnp.tile(b1, 8).reshape(1, 128))
    a2 = _sc_aggregate(xs2.reshape(NPAD, H), src3, dst3, z2)
    zf = _tc_z(a2.reshape(NC, NF, 128), xs2, norm)
    z16 = zf[0:NFV].reshape(N, H)
    return _tc_out(z16, W2, b2.reshape(1, C)).T


# speedup vs baseline: 90.9470x; 1.0189x over previous
import functools

import jax
import jax.numpy as jnp
from jax import lax
from jax.experimental import pallas as pl
from jax.experimental.pallas import tpu as pltpu
from jax.experimental.pallas import tpu_sc as plsc

N = 10000
E = 320000
F_IN = 128
H = 16
C = 100

NC = 2
NS = 16
NW = NC * NS
BB = 128
NB = 79
EPT = NB * BB
EPAD = NW * EPT - E
PADN = 10200
NPAD = 10240
RPS = NPAD // NS
BLK = 2000
DEPTH = 4

_mesh = plsc.VectorSubcoreMesh(core_axis_name="c", subcore_axis_name="s")
_sc_params = pltpu.CompilerParams(use_tc_tiling_on_sc=False)


@functools.partial(
    pl.kernel,
    out_type=jax.ShapeDtypeStruct((NC, NPAD, H), jnp.float32),
    mesh=_mesh,
    scratch_types=[
        pltpu.VMEM((NB, BB), jnp.int32),
        pltpu.VMEM((BB,), jnp.float32),
        pltpu.VMEM((RPS,), jnp.float32),
        pltpu.VMEM((RPS, H), jnp.float32),
        pltpu.VMEM_SHARED((NPAD,), jnp.float32),
        pltpu.SemaphoreType.DMA,
        pltpu.SemaphoreType.DMA,
    ],
    compiler_params=_sc_params,
)
def _sc_degree(dst_hbm, ones_hbm, zeros_hbm, out_hbm,
               idx_v, ones_v, buf_v, rep_v, tbl, sem, isem):
    c = lax.axis_index("c")
    s = lax.axis_index("s")
    wid = c * NS + s

    idx = pltpu.async_copy(dst_hbm.at[wid], idx_v, isem)
    pltpu.sync_copy(ones_hbm, ones_v)
    pltpu.sync_copy(zeros_hbm, buf_v)
    pltpu.sync_copy(buf_v, tbl.at[pl.ds(s * RPS, RPS)])
    idx.wait()
    plsc.subcore_barrier()

    def _fire(j, _):
        pltpu.async_copy(ones_v, tbl.at[idx_v.at[j]], sem, add=True)
        return 0

    lax.fori_loop(0, NB, _fire, 0)

    def _drain(j, _):
        pltpu.make_async_copy(ones_v, tbl.at[idx_v.at[0]], sem).wait()
        return 0

    lax.fori_loop(0, NB, _drain, 0)

    plsc.subcore_barrier()
    pltpu.sync_copy(tbl.at[pl.ds(s * RPS, RPS)], buf_v)

    def _rep(i, _):
        v = buf_v[pl.ds(i * 16, 16)]
        for k in range(16):
            rep_v[i * 16 + k] = jnp.full((H,), v[k], jnp.float32)
        return 0

    lax.fori_loop(0, RPS // 16, _rep, 0)
    pltpu.sync_copy(rep_v, out_hbm.at[c, pl.ds(s * RPS, RPS)])


@functools.partial(
    pl.kernel,
    out_type=jax.ShapeDtypeStruct((NC, NPAD, H), jnp.float32),
    mesh=_mesh,
    scratch_types=(
        [
            pltpu.VMEM((NB, BB), jnp.int32),
            pltpu.VMEM((NB, BB), jnp.int32),
            pltpu.VMEM((DEPTH, BB, H), jnp.float32),
            pltpu.VMEM((RPS, H), jnp.float32),
            pltpu.VMEM((RPS, H), jnp.float32),
            pltpu.VMEM_SHARED((NPAD, H), jnp.float32),
            pltpu.VMEM_SHARED((NPAD, H), jnp.float32),
        ]
        + [pltpu.SemaphoreType.DMA] * (2 * DEPTH + 2)
    ),
    compiler_params=_sc_params,
)
def _sc_aggregate(xs_hbm, src_hbm, dst_hbm, zeros_hbm, out_hbm,
                  src_v, dst_v, rows_v, zbuf_v, sbuf_v, xs_sh, tbl,
                  g0, g1, g2, g3, s0, s1, s2, s3, isem, stg):
    c = lax.axis_index("c")
    s = lax.axis_index("s")
    wid = c * NS + s
    gsem = (g0, g1, g2, g3)
    ssem = (s0, s1, s2, s3)

    pltpu.async_copy(src_hbm.at[wid], src_v, isem)
    stage = pltpu.async_copy(xs_hbm.at[pl.ds(s * RPS, RPS)], sbuf_v, stg)
    pltpu.async_copy(dst_hbm.at[wid], dst_v, isem)

    pltpu.sync_copy(zeros_hbm, zbuf_v)
    pltpu.sync_copy(zbuf_v, tbl.at[pl.ds(s * RPS, RPS)])
    stage.wait()
    pltpu.sync_copy(sbuf_v, xs_sh.at[pl.ds(s * RPS, RPS)])

    pltpu.make_async_copy(src_hbm.at[wid], src_v, isem).wait()
    pltpu.make_async_copy(dst_hbm.at[wid], dst_v, isem).wait()
    plsc.subcore_barrier()

    def _gather(j, b):
        pltpu.async_copy(xs_sh.at[src_v.at[j]], rows_v.at[b], gsem[b])

    def _gwait(b):
        pltpu.make_async_copy(xs_sh.at[src_v.at[0]],
                              rows_v.at[b], gsem[b]).wait()

    def _scatter(j, b):
        pltpu.async_copy(rows_v.at[b], tbl.at[dst_v.at[j]], ssem[b], add=True)

    def _swait(b):
        pltpu.make_async_copy(
            rows_v.at[b], tbl.at[dst_v.at[0]], ssem[b]).wait()

    for b in range(DEPTH):
        _gather(b, b)

    def _round(i, _):
        for b in range(DEPTH):
            j = DEPTH * i + b
            _gwait(b)
            _scatter(j, b)
            _swait(b)
            _gather(j + DEPTH, b)
        return 0

    lax.fori_loop(0, 18, _round, 0)
    for b in range(DEPTH):
        j = 72 + b
        _gwait(b)
        _scatter(j, b)
        _swait(b)
        if j + DEPTH < NB:
            _gather(j + DEPTH, b)
    for b in range(3):
        _gwait(b)
        _scatter(76 + b, b)
        _swait(b)

    plsc.subcore_barrier()
    pltpu.sync_copy(tbl.at[pl.ds(s * RPS, RPS)], zbuf_v)
    pltpu.sync_copy(zbuf_v, out_hbm.at[c, pl.ds(s * RPS, RPS)])


def _tc_edges_body(e_ref, src_ref, dst_ref):
    src_ref[pl.ds(0, E)] = e_ref[0]
    dst_ref[pl.ds(0, E)] = e_ref[1]
    padv = jnp.full((EPAD,), PADN, jnp.int32)
    src_ref[pl.ds(E, EPAD)] = padv
    dst_ref[pl.ds(E, EPAD)] = padv


_tc_edges = pl.pallas_call(
    _tc_edges_body,
    out_shape=[
        jax.ShapeDtypeStruct((NW * NB * BB,), jnp.int32),
        jax.ShapeDtypeStruct((NW * NB * BB,), jnp.int32),
    ],
)


NF = NPAD * H // 128
FB = NF // 5
NFV = N * H // 128
FBV = NFV // 5


def _tc_matmul_body(x8_ref, w1_ref, hw_ref):
    w1 = w1_ref[...]
    blocks = [
        jnp.pad(w1, ((0, 0), (16 * k, 112 - 16 * k))) for k in range(8)
    ]
    w1big = jnp.concatenate(blocks, axis=0)
    hw_ref[pl.ds(0, NFV), :] = jnp.dot(x8_ref[...], w1big,
                                       preferred_element_type=jnp.float32)


_tc_matmul = pl.pallas_call(
    _tc_matmul_body,
    out_shape=jax.ShapeDtypeStruct((NF, 128), jnp.float32),
)


def _tc_scale_body(deg_ref, hw_ref, norm_ref, xs1_ref):
    nrm = lax.rsqrt(1.0 + deg_ref[0] + deg_ref[1])
    norm_ref[...] = nrm
    xs1_ref[...] = hw_ref[...] * nrm


_tc_scale = pl.pallas_call(
    _tc_scale_body,
    grid=(5,),
    in_specs=[
        pl.BlockSpec((NC, FB, 128), lambda j: (0, j, 0)),
        pl.BlockSpec((FB, 128), lambda j: (j, 0)),
    ],
    out_specs=[
        pl.BlockSpec((FB, 128), lambda j: (j, 0)),
        pl.BlockSpec((FB, 128), lambda j: (j, 0)),
    ],
    out_shape=[
        jax.ShapeDtypeStruct((NF, 128), jnp.float32),
        jax.ShapeDtypeStruct((NF, 128), jnp.float32),
    ],
)


def _tc_mid_body(agg_ref, xs1_ref, norm_ref, b1_ref, xs2_ref):
    nrm = norm_ref[...]
    tot = agg_ref[0] + agg_ref[1] + xs1_ref[...]
    pre = tot * nrm + b1_ref[...]
    xs2_ref[...] = jnp.maximum(pre, 0.0) * nrm


_tc_mid = pl.pallas_call(
    _tc_mid_body,
    grid=(5,),
    in_specs=[
        pl.BlockSpec((NC, FB, 128), lambda j: (0, j, 0)),
        pl.BlockSpec((FB, 128), lambda j: (j, 0)),
        pl.BlockSpec((FB, 128), lambda j: (j, 0)),
        pl.BlockSpec((1, 128), lambda j: (0, 0)),
    ],
    out_specs=pl.BlockSpec((FB, 128), lambda j: (j, 0)),
    out_shape=jax.ShapeDtypeStruct((NF, 128), jnp.float32),
)


def _tc_z_body(agg_ref, xs2_ref, norm_ref, z_ref):
    z_ref[...] = (agg_ref[0] + agg_ref[1] + xs2_ref[...]) * norm_ref[...]


_tc_z = pl.pallas_call(
    _tc_z_body,
    grid=(5,),
    in_specs=[
        pl.BlockSpec((NC, FB, 128), lambda j: (0, j, 0)),
        pl.BlockSpec((FB, 128), lambda j: (j, 0)),
        pl.BlockSpec((FB, 128), lambda j: (j, 0)),
    ],
    out_specs=pl.BlockSpec((FB, 128), lambda j: (j, 0)),
    out_shape=jax.ShapeDtypeStruct((NF, 128), jnp.float32),
)


def _tc_out_body(z_ref, w2_ref, b2_ref, out_ref):
    logits = jnp.dot(z_ref[...], w2_ref[...],
                     preferred_element_type=jnp.float32)
    logits = logits + b2_ref[...]
    m = jnp.max(logits, axis=1, keepdims=True)
    shifted = logits - m
    lse = jnp.log(jnp.sum(jnp.exp(shifted), axis=1, keepdims=True))
    out_ref[...] = (shifted - lse).T


_tc_out = pl.pallas_call(
    _tc_out_body,
    out_shape=jax.ShapeDtypeStruct((C, N), jnp.float32),
)


def kernel(x, edge_index, W1, b1, W2, b2):
    ones1 = jnp.ones((BB,), jnp.float32)
    z1 = jnp.zeros((RPS,), jnp.float32)
    z2 = jnp.zeros((RPS, H), jnp.float32)

    src1, dst1 = _tc_edges(edge_index)
    src3 = src1.reshape(NW, NB, BB)
    dst3 = dst1.reshape(NW, NB, BB)
    degp = _sc_degree(dst3, ones1, z1)
    hw = _tc_matmul(x.reshape(N // 8, 1024), W1)
    norm, xs1 = _tc_scale(degp.reshape(NC, NF, 128), hw)
    a1 = _sc_aggregate(xs1.reshape(NPAD, H), src3, dst3, z2)
    xs2 = _tc_mid(a1.reshape(NC, NF, 128), xs1, norm,
                  jnp.tile(b1, 8).reshape(1, 128))
    a2 = _sc_aggregate(xs2.reshape(NPAD, H), src3, dst3, z2)
    zf = _tc_z(a2.reshape(NC, NF, 128), xs2, norm)
    z16 = zf[0:NFV].reshape(N, H)
    return _tc_out(z16, W2, b2.reshape(1, C)).T
